# baseline (speedup 1.0000x reference)
import functools

import jax
import jax.numpy as jnp
from jax import lax
from jax.experimental import pallas as pl
from jax.experimental.pallas import tpu as pltpu
from jax.experimental.pallas import tpu_sc as plsc

N_NODES = 10000
NP = 10240
D = 128
C_OUT = 10
CP = 16
E_RAW = 320000
CAPV = 1344
REG = CAPV * 16
E_BUCK = 256 * CAPV
CH = 14336
NCHUNK = E_BUCK // CH
NTILES = 32
FPT = D // NTILES
ESH = E_BUCK // NTILES
COLB = 1024

_mesh = plsc.VectorSubcoreMesh(core_axis_name="c", subcore_axis_name="s")
_sc_params = pltpu.CompilerParams(needs_layout_passes=False)


def _wid():
    return lax.axis_index("c") * 16 + lax.axis_index("s")


def _unpack(pk):
    s16 = lax.bitwise_and(pk, jnp.int32(0xFFFF))
    d16 = lax.shift_right_logical(pk, jnp.int32(16))
    return s16, d16


@functools.partial(
    pl.kernel,
    out_type=jax.ShapeDtypeStruct((NTILES, NP), jnp.float32),
    mesh=_mesh,
    compiler_params=_sc_params,
    scratch_types=[
        pltpu.VMEM((NP,), jnp.float32),
        pltpu.VMEM((ESH,), jnp.int32),
        pltpu.SemaphoreType.DMA,
    ],
)
def _deg_kernel(edge_hbm, hist_hbm, hist_t, ebuf, sem):
    wid = _wid()
    pltpu.async_copy(edge_hbm.at[pl.ds(wid * ESH, ESH)], ebuf, sem).wait()

    @pl.loop(0, NP, step=16, unroll=4)
    def _(i):
        hist_t[pl.ds(i, 16)] = jnp.zeros((16,), jnp.float32)

    one16 = jnp.ones((16,), jnp.float32)

    @plsc.parallel_loop(0, ESH, 16, unroll=4)
    def _(e):
        pk = ebuf[pl.ds(e, 16)]
        _, d16 = _unpack(pk)
        plsc.addupdate_scatter(hist_t, [d16], one16)

    pltpu.sync_copy(hist_t, hist_hbm.at[wid])


@functools.partial(
    pl.kernel,
    out_type=jax.ShapeDtypeStruct((D, NP), jnp.float32),
    mesh=_mesh,
    compiler_params=_sc_params,
    scratch_types=[
        [pltpu.VMEM((NP,), jnp.float32)] * FPT,
        [pltpu.VMEM((NP,), jnp.float32)] * FPT,
        pltpu.VMEM((NP,), jnp.float32),
        pltpu.VMEM((2, CH), jnp.int32),
        pltpu.VMEM((16,), jnp.int32),
        pltpu.SemaphoreType.DMA,
        pltpu.SemaphoreType.DMA,
        pltpu.SemaphoreType.DMA,
    ],
)
def _prop_kernel(u0_hbm, edge_hbm, dinv_hbm, ns_hbm, out_hbm,
                 u_refs, acc_refs, dinv_t, ebuf, nsv, sem_a, sem_b, sem_m):
    wid = _wid()
    f0 = wid * FPT
    for f in range(FPT):
        pltpu.async_copy(u0_hbm.at[f0 + f], u_refs[f], sem_m)
    pltpu.async_copy(dinv_hbm, dinv_t, sem_m)
    pltpu.async_copy(ns_hbm, nsv, sem_m)
    for f in range(FPT):
        pltpu.make_async_copy(u0_hbm.at[f0 + f], u_refs[f], sem_m).wait()
    pltpu.make_async_copy(dinv_hbm, dinv_t, sem_m).wait()
    pltpu.make_async_copy(ns_hbm, nsv, sem_m).wait()
    nsteps = jnp.max(nsv[...])

    def _issue(ci, buf, sem):
        pltpu.async_copy(edge_hbm.at[pl.ds(ci * CH, CH)], ebuf.at[buf], sem)

    def _wait(ci, buf, sem):
        pltpu.make_async_copy(
            edge_hbm.at[pl.ds(ci * CH, CH)], ebuf.at[buf], sem).wait()

    def _process(buf):
        @plsc.parallel_loop(0, CH, 16, unroll=4)
        def _(e):
            pk = ebuf[buf, pl.ds(e, 16)]
            s16, d16 = _unpack(pk)
            vs = [plsc.load_gather(u_refs[f], [s16]) for f in range(FPT)]
            for f in range(FPT):
                plsc.addupdate_scatter(acc_refs[f], [d16], vs[f])

    def _step(_, carry):
        @pl.loop(0, NP, step=16, unroll=4)
        def _(i):
            for f in range(FPT):
                acc_refs[f][pl.ds(i, 16)] = u_refs[f][pl.ds(i, 16)]

        _issue(0, 0, sem_a)

        @pl.loop(0, NCHUNK, step=2)
        def _(ci):
            _issue(ci + 1, 1, sem_b)
            _wait(ci, 0, sem_a)
            _process(0)

            @pl.when(ci + 2 < NCHUNK)
            def _():
                _issue(ci + 2, 0, sem_a)

            _wait(ci + 1, 1, sem_b)
            _process(1)

        @pl.loop(0, NP, step=16, unroll=4)
        def _(i):
            dv = dinv_t[pl.ds(i, 16)]
            for f in range(FPT):
                u_refs[f][pl.ds(i, 16)] = acc_refs[f][pl.ds(i, 16)] * dv

        return carry

    lax.fori_loop(0, nsteps, _step, 0)
    for f in range(FPT):
        pltpu.async_copy(u_refs[f], out_hbm.at[f0 + f], sem_m)
    for f in range(FPT):
        pltpu.make_async_copy(u_refs[f], out_hbm.at[f0 + f], sem_m).wait()


@functools.partial(
    pl.kernel,
    out_type=jax.ShapeDtypeStruct((CP, NP), jnp.float32),
    mesh=_mesh,
    compiler_params=_sc_params,
    scratch_types=[
        pltpu.VMEM((NP,), jnp.float32),
        pltpu.VMEM((NP,), jnp.float32),
        pltpu.VMEM((NP,), jnp.float32),
        pltpu.VMEM((2, CH), jnp.int32),
        pltpu.SemaphoreType.DMA,
        pltpu.SemaphoreType.DMA,
        pltpu.SemaphoreType.DMA,
    ],
)
def _final_kernel(z_hbm, edge_hbm, disq_hbm, out_hbm,
                  z_t, acc_t, disq_t, ebuf, sem_a, sem_b, sem_m):
    wid = _wid()

    @pl.when(wid < CP)
    def _():
        pltpu.async_copy(z_hbm.at[wid], z_t, sem_m).wait()
        pltpu.async_copy(disq_hbm, disq_t, sem_m).wait()

        @pl.loop(0, NP, step=16, unroll=4)
        def _(i):
            acc_t[pl.ds(i, 16)] = z_t[pl.ds(i, 16)]

        def _issue(ci, buf, sem):
            pltpu.async_copy(edge_hbm.at[pl.ds(ci * CH, CH)], ebuf.at[buf], sem)

        def _wait(ci, buf, sem):
            pltpu.make_async_copy(
                edge_hbm.at[pl.ds(ci * CH, CH)], ebuf.at[buf], sem).wait()

        def _process(buf):
            @plsc.parallel_loop(0, CH, 16, unroll=4)
            def _(e):
                pk = ebuf[buf, pl.ds(e, 16)]
                s16, d16 = _unpack(pk)
                v = plsc.load_gather(z_t, [s16])
                plsc.addupdate_scatter(acc_t, [d16], v)

        _issue(0, 0, sem_a)

        @pl.loop(0, NCHUNK, step=2)
        def _(ci):
            _issue(ci + 1, 1, sem_b)
            _wait(ci, 0, sem_a)
            _process(0)

            @pl.when(ci + 2 < NCHUNK)
            def _():
                _issue(ci + 2, 0, sem_a)

            _wait(ci + 1, 1, sem_b)
            _process(1)

        @pl.loop(0, NP, step=16, unroll=4)
        def _(i):
            acc_t[pl.ds(i, 16)] = acc_t[pl.ds(i, 16)] * disq_t[pl.ds(i, 16)]

        pltpu.sync_copy(acc_t, out_hbm.at[wid])


def _mm0_body(w_ref, x_ref, b_ref, o_ref):
    o_ref[...] = lax.dot_general(
        w_ref[...], x_ref[...], (((0,), (1,)), ((), ())),
        preferred_element_type=jnp.float32) + b_ref[...]


_mm0 = pl.pallas_call(
    _mm0_body,
    grid=(NP // COLB,),
    in_specs=[
        pl.BlockSpec((D, D), lambda i: (0, 0)),
        pl.BlockSpec((COLB, D), lambda i: (i, 0)),
        pl.BlockSpec((D, 1), lambda i: (0, 0)),
    ],
    out_specs=pl.BlockSpec((D, COLB), lambda i: (0, i)),
    out_shape=jax.ShapeDtypeStruct((D, NP), jnp.float32),
)


def _scale_body(h_ref, hist_ref, u0_ref, dinv_ref, dsq_ref, disq_ref):
    deg = 1.0 + jnp.sum(hist_ref[...], axis=0, keepdims=True)
    pos = deg > 0
    dinv_ref[...] = jnp.where(pos, 1.0 / deg, 0.0)
    sq = jnp.sqrt(deg)
    dsq_ref[...] = sq
    disq = jnp.where(pos, 1.0 / sq, 0.0)
    disq_ref[...] = disq
    u0_ref[...] = h_ref[...] * disq


_scale = pl.pallas_call(
    _scale_body,
    grid=(NP // COLB,),
    in_specs=[
        pl.BlockSpec((D, COLB), lambda i: (0, i)),
        pl.BlockSpec((NTILES, COLB), lambda i: (0, i)),
    ],
    out_specs=[
        pl.BlockSpec((D, COLB), lambda i: (0, i)),
        pl.BlockSpec((1, COLB), lambda i: (0, i)),
        pl.BlockSpec((1, COLB), lambda i: (0, i)),
        pl.BlockSpec((1, COLB), lambda i: (0, i)),
    ],
    out_shape=[
        jax.ShapeDtypeStruct((D, NP), jnp.float32),
        jax.ShapeDtypeStruct((1, NP), jnp.float32),
        jax.ShapeDtypeStruct((1, NP), jnp.float32),
        jax.ShapeDtypeStruct((1, NP), jnp.float32),
    ],
)


def _clf_body(wt_ref, u_ref, dsq_ref, disq_ref, bc_ref, z_ref):
    y = jnp.maximum(u_ref[...], 0.0) * dsq_ref[...]
    z = lax.dot_general(
        wt_ref[...], y, (((1,), (0,)), ((), ())),
        preferred_element_type=jnp.float32)
    z_ref[...] = (z + bc_ref[...]) * disq_ref[...]


_clf = pl.pallas_call(
    _clf_body,
    grid=(NP // COLB,),
    in_specs=[
        pl.BlockSpec((CP, D), lambda i: (0, 0)),
        pl.BlockSpec((D, COLB), lambda i: (0, i)),
        pl.BlockSpec((1, COLB), lambda i: (0, i)),
        pl.BlockSpec((1, COLB), lambda i: (0, i)),
        pl.BlockSpec((CP, 1), lambda i: (0, 0)),
    ],
    out_specs=pl.BlockSpec((CP, COLB), lambda i: (0, i)),
    out_shape=jax.ShapeDtypeStruct((CP, NP), jnp.float32),
)


def kernel(x, edge_index, prop_nums, W0, b0, Wc, bc):
    src = edge_index[0].astype(jnp.int32)
    dst = edge_index[1].astype(jnp.int32)
    pk = jnp.bitwise_or(src, jnp.left_shift(dst, 16))
    lane = src % 16
    kcl = (dst - src) % 16
    b = kcl * 16 + lane
    order = jnp.argsort(b, stable=False)
    bs = b[order]
    pks = pk[order]
    ii = jnp.arange(E_RAW, dtype=jnp.int32)
    run_start = jnp.concatenate([jnp.ones((1,), jnp.bool_), bs[1:] != bs[:-1]])
    starts = lax.cummax(jnp.where(run_start, ii, 0))
    r = ii - starts
    in_cap = r < CAPV
    pos_main = (bs // 16) * REG + r * 16 + (bs % 16)
    usedi = jnp.zeros((E_BUCK + 1,), jnp.int32).at[
        jnp.where(in_cap, pos_main, E_BUCK)].add(1)
    free = usedi[:E_BUCK] == 0
    fsl = jnp.cumsum(free.astype(jnp.int32)) - 1
    free_tab = jnp.zeros((E_BUCK + 1,), jnp.int32).at[
        jnp.where(free, fsl, E_BUCK)].set(jnp.arange(E_BUCK, dtype=jnp.int32))
    spill_rank = jnp.cumsum((~in_cap).astype(jnp.int32)) - 1
    pos = jnp.where(in_cap, pos_main, free_tab[spill_rank])
    t = jnp.arange(E_BUCK, dtype=jnp.int32)
    lpad = t % 16
    kpad = t // REG
    spad = N_NODES + lpad
    dpad = N_NODES + ((lpad + kpad) % 16)
    pad_pk = jnp.bitwise_or(spad, jnp.left_shift(dpad, 16))
    epk = pad_pk.at[pos].set(pks)

    x_pad = jnp.pad(x, ((0, NP - N_NODES), (0, 0)))
    b0c = b0.reshape(D, 1)
    wct = jnp.pad(Wc, ((0, 0), (0, CP - C_OUT))).T
    bcp = jnp.pad(bc, (0, CP - C_OUT)).reshape(CP, 1)
    ns_arr = jnp.full((16,), prop_nums, jnp.int32)

    hist = _deg_kernel(epk)
    h0t = _mm0(W0, x_pad, b0c)
    u0, dinv, dsq, disq = _scale(h0t, hist)
    u30 = _prop_kernel(u0, epk, dinv.reshape(NP), ns_arr)
    z2 = _clf(wct, u30, dsq, disq, bcp)
    outt = _final_kernel(z2, epk, disq.reshape(NP))
    return outt[:C_OUT, :N_NODES].T


# speedup vs baseline: 1.3713x; 1.3713x over previous
import functools

import jax
import jax.numpy as jnp
from jax import lax
from jax.experimental import pallas as pl
from jax.experimental.pallas import tpu as pltpu
from jax.experimental.pallas import tpu_sc as plsc

N_NODES = 10000
NP = 10240
D = 128
C_OUT = 10
CP = 16
E_RAW = 320000
CAPV = 1344
REG = CAPV * 16
E_BUCK = 256 * CAPV
E_SPARE = 16384
E_MAIN = E_BUCK + E_SPARE
CH = 16384
NCHUNK = E_MAIN // CH
NCH3 = 19
E_TOT = E_MAIN + NCH3 * CH
NTILES = 32
FPT = D // NTILES
ESH = E_TOT // NTILES
COLB = 1024

_mesh = plsc.VectorSubcoreMesh(core_axis_name="c", subcore_axis_name="s")
_sc_params = pltpu.CompilerParams(needs_layout_passes=False)


def _wid():
    return lax.axis_index("c") * 16 + lax.axis_index("s")


def _unpack(pk):
    s16 = lax.bitwise_and(pk, jnp.int32(0xFFFF))
    d16 = lax.shift_right_logical(pk, jnp.int32(16))
    return s16, d16


@functools.partial(
    pl.kernel,
    out_type=jax.ShapeDtypeStruct((NTILES, NP), jnp.float32),
    mesh=_mesh,
    compiler_params=_sc_params,
    scratch_types=[
        pltpu.VMEM((NP,), jnp.float32),
        pltpu.VMEM((ESH,), jnp.int32),
        pltpu.SemaphoreType.DMA,
    ],
)
def _deg_kernel(edge_hbm, hist_hbm, hist_t, ebuf, sem):
    wid = _wid()
    pltpu.async_copy(edge_hbm.at[pl.ds(wid * ESH, ESH)], ebuf, sem).wait()

    @pl.loop(0, NP, step=16, unroll=4)
    def _(i):
        hist_t[pl.ds(i, 16)] = jnp.zeros((16,), jnp.float32)

    one16 = jnp.ones((16,), jnp.float32)

    @plsc.parallel_loop(0, ESH, 16, unroll=4)
    def _(e):
        pk = ebuf[pl.ds(e, 16)]
        _, d16 = _unpack(pk)
        plsc.addupdate_scatter(hist_t, [d16], one16)

    pltpu.sync_copy(hist_t, hist_hbm.at[wid])


@functools.partial(
    pl.kernel,
    out_type=jax.ShapeDtypeStruct((D, NP), jnp.float32),
    mesh=_mesh,
    compiler_params=_sc_params,
    scratch_types=[
        [pltpu.VMEM((NP,), jnp.float32)] * FPT,
        [pltpu.VMEM((NP,), jnp.float32)] * FPT,
        pltpu.VMEM((NP,), jnp.float32),
        pltpu.VMEM((2, CH), jnp.int32),
        pltpu.VMEM((16,), jnp.int32),
        pltpu.SemaphoreType.DMA,
        pltpu.SemaphoreType.DMA,
        pltpu.SemaphoreType.DMA,
    ],
)
def _prop_kernel(u0_hbm, edge_hbm, dinv_hbm, ns_hbm, out_hbm,
                 u_refs, acc_refs, dinv_t, ebuf, nsv, sem_a, sem_b, sem_m):
    wid = _wid()
    f0 = wid * FPT
    for f in range(FPT):
        pltpu.async_copy(u0_hbm.at[f0 + f], u_refs[f], sem_m)
    pltpu.async_copy(dinv_hbm, dinv_t, sem_m)
    pltpu.async_copy(ns_hbm, nsv, sem_m)
    for f in range(FPT):
        pltpu.make_async_copy(u0_hbm.at[f0 + f], u_refs[f], sem_m).wait()
    pltpu.make_async_copy(dinv_hbm, dinv_t, sem_m).wait()
    pltpu.make_async_copy(ns_hbm, nsv, sem_m).wait()
    nspk = jnp.max(nsv[...])
    nsteps = lax.bitwise_and(nspk, jnp.int32(0xFF))
    nch3 = lax.shift_right_logical(nspk, jnp.int32(8))

    def _issue(ci, buf, sem):
        pltpu.async_copy(edge_hbm.at[pl.ds(ci * CH, CH)], ebuf.at[buf], sem)

    def _wait(ci, buf, sem):
        pltpu.make_async_copy(
            edge_hbm.at[pl.ds(ci * CH, CH)], ebuf.at[buf], sem).wait()

    def _process(buf):
        @plsc.parallel_loop(0, CH, 16, unroll=4)
        def _(e):
            pk = ebuf[buf, pl.ds(e, 16)]
            s16, d16 = _unpack(pk)
            vs = [plsc.load_gather(u_refs[f], [s16]) for f in range(FPT)]
            for f in range(FPT):
                plsc.addupdate_scatter(acc_refs[f], [d16], vs[f])

    def _step(_, carry):
        @pl.loop(0, NP, step=16, unroll=4)
        def _(i):
            for f in range(FPT):
                acc_refs[f][pl.ds(i, 16)] = u_refs[f][pl.ds(i, 16)]

        _issue(0, 0, sem_a)

        @pl.loop(0, NCHUNK, step=2)
        def _(ci):
            _issue(ci + 1, 1, sem_b)
            _wait(ci, 0, sem_a)
            _process(0)

            @pl.when(ci + 2 < NCHUNK)
            def _():
                _issue(ci + 2, 0, sem_a)

            _wait(ci + 1, 1, sem_b)
            _process(1)

        @pl.loop(0, nch3)
        def _(c3):
            pltpu.sync_copy(edge_hbm.at[pl.ds(E_MAIN + c3 * CH, CH)],
                            ebuf.at[0])
            _process(0)

        @pl.loop(0, NP, step=16, unroll=4)
        def _(i):
            dv = dinv_t[pl.ds(i, 16)]
            for f in range(FPT):
                u_refs[f][pl.ds(i, 16)] = acc_refs[f][pl.ds(i, 16)] * dv

        return carry

    lax.fori_loop(0, nsteps, _step, 0)
    for f in range(FPT):
        pltpu.async_copy(u_refs[f], out_hbm.at[f0 + f], sem_m)
    for f in range(FPT):
        pltpu.make_async_copy(u_refs[f], out_hbm.at[f0 + f], sem_m).wait()


@functools.partial(
    pl.kernel,
    out_type=jax.ShapeDtypeStruct((CP, NP), jnp.float32),
    mesh=_mesh,
    compiler_params=_sc_params,
    scratch_types=[
        pltpu.VMEM((NP,), jnp.float32),
        pltpu.VMEM((NP,), jnp.float32),
        pltpu.VMEM((NP,), jnp.float32),
        pltpu.VMEM((2, CH), jnp.int32),
        pltpu.VMEM((16,), jnp.int32),
        pltpu.SemaphoreType.DMA,
        pltpu.SemaphoreType.DMA,
        pltpu.SemaphoreType.DMA,
    ],
)
def _final_kernel(z_hbm, edge_hbm, disq_hbm, ns_hbm, out_hbm,
                  z_t, acc_t, disq_t, ebuf, nsv, sem_a, sem_b, sem_m):
    wid = _wid()

    @pl.when(wid < CP)
    def _():
        pltpu.async_copy(z_hbm.at[wid], z_t, sem_m).wait()
        pltpu.async_copy(disq_hbm, disq_t, sem_m).wait()
        pltpu.async_copy(ns_hbm, nsv, sem_m).wait()
        nch3 = lax.shift_right_logical(jnp.max(nsv[...]), jnp.int32(8))

        @pl.loop(0, NP, step=16, unroll=4)
        def _(i):
            acc_t[pl.ds(i, 16)] = z_t[pl.ds(i, 16)]

        def _issue(ci, buf, sem):
            pltpu.async_copy(edge_hbm.at[pl.ds(ci * CH, CH)], ebuf.at[buf], sem)

        def _wait(ci, buf, sem):
            pltpu.make_async_copy(
                edge_hbm.at[pl.ds(ci * CH, CH)], ebuf.at[buf], sem).wait()

        def _process(buf):
            @plsc.parallel_loop(0, CH, 16, unroll=4)
            def _(e):
                pk = ebuf[buf, pl.ds(e, 16)]
                s16, d16 = _unpack(pk)
                v = plsc.load_gather(z_t, [s16])
                plsc.addupdate_scatter(acc_t, [d16], v)

        _issue(0, 0, sem_a)

        @pl.loop(0, NCHUNK, step=2)
        def _(ci):
            _issue(ci + 1, 1, sem_b)
            _wait(ci, 0, sem_a)
            _process(0)

            @pl.when(ci + 2 < NCHUNK)
            def _():
                _issue(ci + 2, 0, sem_a)

            _wait(ci + 1, 1, sem_b)
            _process(1)

        @pl.loop(0, nch3)
        def _(c3):
            pltpu.sync_copy(edge_hbm.at[pl.ds(E_MAIN + c3 * CH, CH)],
                            ebuf.at[0])
            _process(0)

        @pl.loop(0, NP, step=16, unroll=4)
        def _(i):
            acc_t[pl.ds(i, 16)] = acc_t[pl.ds(i, 16)] * disq_t[pl.ds(i, 16)]

        pltpu.sync_copy(acc_t, out_hbm.at[wid])


def _mm0_body(w_ref, x_ref, b_ref, o_ref):
    o_ref[...] = lax.dot_general(
        w_ref[...], x_ref[...], (((0,), (1,)), ((), ())),
        preferred_element_type=jnp.float32) + b_ref[...]


_mm0 = pl.pallas_call(
    _mm0_body,
    grid=(NP // COLB,),
    in_specs=[
        pl.BlockSpec((D, D), lambda i: (0, 0)),
        pl.BlockSpec((COLB, D), lambda i: (i, 0)),
        pl.BlockSpec((D, 1), lambda i: (0, 0)),
    ],
    out_specs=pl.BlockSpec((D, COLB), lambda i: (0, i)),
    out_shape=jax.ShapeDtypeStruct((D, NP), jnp.float32),
)


def _scale_body(h_ref, hist_ref, u0_ref, dinv_ref, dsq_ref, disq_ref):
    deg = 1.0 + jnp.sum(hist_ref[...], axis=0, keepdims=True)
    pos = deg > 0
    dinv_ref[...] = jnp.where(pos, 1.0 / deg, 0.0)
    sq = jnp.sqrt(deg)
    dsq_ref[...] = sq
    disq = jnp.where(pos, 1.0 / sq, 0.0)
    disq_ref[...] = disq
    u0_ref[...] = h_ref[...] * disq


_scale = pl.pallas_call(
    _scale_body,
    grid=(NP // COLB,),
    in_specs=[
        pl.BlockSpec((D, COLB), lambda i: (0, i)),
        pl.BlockSpec((NTILES, COLB), lambda i: (0, i)),
    ],
    out_specs=[
        pl.BlockSpec((D, COLB), lambda i: (0, i)),
        pl.BlockSpec((1, COLB), lambda i: (0, i)),
        pl.BlockSpec((1, COLB), lambda i: (0, i)),
        pl.BlockSpec((1, COLB), lambda i: (0, i)),
    ],
    out_shape=[
        jax.ShapeDtypeStruct((D, NP), jnp.float32),
        jax.ShapeDtypeStruct((1, NP), jnp.float32),
        jax.ShapeDtypeStruct((1, NP), jnp.float32),
        jax.ShapeDtypeStruct((1, NP), jnp.float32),
    ],
)


def _clf_body(wt_ref, u_ref, dsq_ref, disq_ref, bc_ref, z_ref):
    y = jnp.maximum(u_ref[...], 0.0) * dsq_ref[...]
    z = lax.dot_general(
        wt_ref[...], y, (((1,), (0,)), ((), ())),
        preferred_element_type=jnp.float32)
    z_ref[...] = (z + bc_ref[...]) * disq_ref[...]


_clf = pl.pallas_call(
    _clf_body,
    grid=(NP // COLB,),
    in_specs=[
        pl.BlockSpec((CP, D), lambda i: (0, 0)),
        pl.BlockSpec((D, COLB), lambda i: (0, i)),
        pl.BlockSpec((1, COLB), lambda i: (0, i)),
        pl.BlockSpec((1, COLB), lambda i: (0, i)),
        pl.BlockSpec((CP, 1), lambda i: (0, 0)),
    ],
    out_specs=pl.BlockSpec((CP, COLB), lambda i: (0, i)),
    out_shape=jax.ShapeDtypeStruct((CP, NP), jnp.float32),
)


def kernel(x, edge_index, prop_nums, W0, b0, Wc, bc):
    src = edge_index[0].astype(jnp.int32)
    dst = edge_index[1].astype(jnp.int32)
    pk = jnp.bitwise_or(src, jnp.left_shift(dst, 16))
    lane = src % 16
    kcl = (dst - src) % 16
    b = kcl * 16 + lane
    order = jnp.argsort(b, stable=False)
    bs = b[order]
    pks = pk[order]
    ii = jnp.arange(E_RAW, dtype=jnp.int32)
    run_start = jnp.concatenate([jnp.ones((1,), jnp.bool_), bs[1:] != bs[:-1]])
    starts = lax.cummax(jnp.where(run_start, ii, 0))
    r = ii - starts
    in_cap = r < CAPV
    pos_main = (bs // 16) * REG + r * 16 + (bs % 16)
    spill_rank = jnp.cumsum((~in_cap).astype(jnp.int32)) - 1
    pos = jnp.where(in_cap, pos_main, E_BUCK + spill_rank)
    n_spill = spill_rank[-1] + 1
    n_ovf = jnp.maximum(n_spill - E_SPARE, 0)
    nch3 = (n_ovf + CH - 1) // CH
    t = jnp.arange(E_TOT, dtype=jnp.int32)
    lpad = t % 16
    kpad = (t // REG) % 16
    spad = N_NODES + lpad
    dpad = N_NODES + ((lpad + kpad) % 16)
    pad_pk = jnp.bitwise_or(spad, jnp.left_shift(dpad, 16))
    epk = pad_pk.at[pos].set(pks)

    x_pad = jnp.pad(x, ((0, NP - N_NODES), (0, 0)))
    b0c = b0.reshape(D, 1)
    wct = jnp.pad(Wc, ((0, 0), (0, CP - C_OUT))).T
    bcp = jnp.pad(bc, (0, CP - C_OUT)).reshape(CP, 1)
    ns_arr = jnp.full((16,), prop_nums + nch3 * 256, jnp.int32)

    hist = _deg_kernel(epk)
    h0t = _mm0(W0, x_pad, b0c)
    u0, dinv, dsq, disq = _scale(h0t, hist)
    u30 = _prop_kernel(u0, epk, dinv.reshape(NP), ns_arr)
    z2 = _clf(wct, u30, dsq, disq, bcp)
    outt = _final_kernel(z2, epk, disq.reshape(NP), ns_arr)
    return outt[:C_OUT, :N_NODES].T


# speedup vs baseline: 1.4427x; 1.0521x over previous
import functools

import jax
import jax.numpy as jnp
from jax import lax
from jax.experimental import pallas as pl
from jax.experimental.pallas import tpu as pltpu
from jax.experimental.pallas import tpu_sc as plsc

N_NODES = 10000
NP = 10240
D = 128
C_OUT = 10
CP = 16
E_RAW = 320000
CAPV = 1344
REG = CAPV * 16
E_BUCK = 256 * CAPV
E_SPARE = 16384
E_MAIN = E_BUCK + E_SPARE
CH = 16384
NCHUNK = E_MAIN // CH
NCH3 = 19
E_TOT = E_MAIN + NCH3 * CH
NTILES = 32
FPT = D // NTILES
ESH = E_TOT // NTILES
COLB = 1024

_mesh = plsc.VectorSubcoreMesh(core_axis_name="c", subcore_axis_name="s")
_sc_params = pltpu.CompilerParams(needs_layout_passes=False)


def _wid():
    return lax.axis_index("c") * 16 + lax.axis_index("s")


def _unpack(pk):
    s16 = lax.bitwise_and(pk, jnp.int32(0xFFFF))
    d16 = lax.shift_right_logical(pk, jnp.int32(16))
    return s16, d16


ESH1 = E_RAW // NTILES


@functools.partial(
    pl.kernel,
    out_type=jax.ShapeDtypeStruct((NTILES, 256), jnp.int32),
    mesh=_mesh,
    compiler_params=_sc_params,
    scratch_types=[
        pltpu.VMEM((256,), jnp.int32),
        pltpu.VMEM((ESH1,), jnp.int32),
        pltpu.SemaphoreType.DMA,
    ],
)
def _cnt_kernel(pk_hbm, cnt_hbm, cnt_t, pkbuf, sem):
    wid = _wid()
    pltpu.async_copy(pk_hbm.at[pl.ds(wid * ESH1, ESH1)], pkbuf, sem).wait()

    @pl.loop(0, 256, step=16)
    def _(i):
        cnt_t[pl.ds(i, 16)] = jnp.zeros((16,), jnp.int32)

    one16 = jnp.ones((16,), jnp.int32)

    @plsc.parallel_loop(0, ESH1, 16, unroll=2)
    def _(e):
        pk16 = pkbuf[pl.ds(e, 16)]
        sl = lax.bitwise_and(pk16, jnp.int32(0xFFFF))
        dl = lax.shift_right_logical(pk16, jnp.int32(16))
        b16 = (lax.bitwise_and(dl - sl, jnp.int32(15)) * 16
               + lax.bitwise_and(sl, jnp.int32(15)))
        plsc.addupdate_scatter(cnt_t, [b16], one16)

    pltpu.sync_copy(cnt_t, cnt_hbm.at[wid])


@functools.partial(
    pl.kernel,
    out_type=jax.ShapeDtypeStruct((E_RAW,), jnp.int32),
    mesh=_mesh,
    compiler_params=_sc_params,
    scratch_types=[
        pltpu.VMEM((256,), jnp.int32),
        pltpu.VMEM((256,), jnp.int32),
        pltpu.VMEM((ESH1,), jnp.int32),
        pltpu.VMEM((ESH1,), jnp.int32),
        pltpu.SemaphoreType.DMA,
    ],
)
def _pos_kernel(pk_hbm, base_hbm, sbase_hbm, pos_hbm,
                cnt_t, scnt_t, pkbuf, posbuf, sem):
    wid = _wid()
    pltpu.async_copy(pk_hbm.at[pl.ds(wid * ESH1, ESH1)], pkbuf, sem).wait()
    pltpu.async_copy(base_hbm.at[wid], cnt_t, sem).wait()
    pltpu.async_copy(sbase_hbm.at[wid], scnt_t, sem).wait()

    @pl.loop(0, ESH1, step=16)
    def _(e):
        pk16 = pkbuf[pl.ds(e, 16)]
        sl = lax.bitwise_and(pk16, jnp.int32(0xFFFF))
        dl = lax.shift_right_logical(pk16, jnp.int32(16))
        b16 = (lax.bitwise_and(dl - sl, jnp.int32(15)) * 16
               + lax.bitwise_and(sl, jnp.int32(15)))
        cum, last = plsc.scan_count(b16)
        old = plsc.load_gather(cnt_t, [b16])
        g = old + cum - 1
        plsc.addupdate_scatter(cnt_t, [b16], cum, mask=last)
        incap = g < CAPV
        posm = (lax.shift_right_logical(b16, jnp.int32(4)) * REG + g * 16
                + lax.bitwise_and(b16, jnp.int32(15)))
        notcap = jnp.logical_not(incap)
        scum, slast = plsc.scan_count(b16, mask=notcap)
        sold = plsc.load_gather(scnt_t, [b16])
        sp = sold + scum - 1
        plsc.addupdate_scatter(scnt_t, [b16], scum,
                               mask=jnp.logical_and(slast, notcap))
        posbuf[pl.ds(e, 16)] = jnp.where(incap, posm, E_BUCK + sp)

    pltpu.sync_copy(posbuf, pos_hbm.at[pl.ds(wid * ESH1, ESH1)])


@functools.partial(
    pl.kernel,
    out_type=jax.ShapeDtypeStruct((NTILES, NP), jnp.float32),
    mesh=_mesh,
    compiler_params=_sc_params,
    scratch_types=[
        pltpu.VMEM((NP,), jnp.float32),
        pltpu.VMEM((ESH,), jnp.int32),
        pltpu.SemaphoreType.DMA,
    ],
)
def _deg_kernel(edge_hbm, hist_hbm, hist_t, ebuf, sem):
    wid = _wid()
    pltpu.async_copy(edge_hbm.at[pl.ds(wid * ESH, ESH)], ebuf, sem).wait()

    @pl.loop(0, NP, step=16, unroll=4)
    def _(i):
        hist_t[pl.ds(i, 16)] = jnp.zeros((16,), jnp.float32)

    one16 = jnp.ones((16,), jnp.float32)

    @plsc.parallel_loop(0, ESH, 16, unroll=4)
    def _(e):
        pk = ebuf[pl.ds(e, 16)]
        _, d16 = _unpack(pk)
        plsc.addupdate_scatter(hist_t, [d16], one16)

    pltpu.sync_copy(hist_t, hist_hbm.at[wid])


@functools.partial(
    pl.kernel,
    out_type=jax.ShapeDtypeStruct((D, NP), jnp.float32),
    mesh=_mesh,
    compiler_params=_sc_params,
    scratch_types=[
        [pltpu.VMEM((NP,), jnp.float32)] * FPT,
        [pltpu.VMEM((NP,), jnp.float32)] * FPT,
        pltpu.VMEM((NP,), jnp.float32),
        pltpu.VMEM((2, CH), jnp.int32),
        pltpu.VMEM((16,), jnp.int32),
        pltpu.SemaphoreType.DMA,
        pltpu.SemaphoreType.DMA,
        pltpu.SemaphoreType.DMA,
    ],
)
def _prop_kernel(u0_hbm, edge_hbm, dinv_hbm, ns_hbm, out_hbm,
                 u_refs, acc_refs, dinv_t, ebuf, nsv, sem_a, sem_b, sem_m):
    wid = _wid()
    f0 = wid * FPT
    for f in range(FPT):
        pltpu.async_copy(u0_hbm.at[f0 + f], u_refs[f], sem_m)
    pltpu.async_copy(dinv_hbm, dinv_t, sem_m)
    pltpu.async_copy(ns_hbm, nsv, sem_m)
    for f in range(FPT):
        pltpu.make_async_copy(u0_hbm.at[f0 + f], u_refs[f], sem_m).wait()
    pltpu.make_async_copy(dinv_hbm, dinv_t, sem_m).wait()
    pltpu.make_async_copy(ns_hbm, nsv, sem_m).wait()
    nspk = jnp.max(nsv[...])
    nsteps = lax.bitwise_and(nspk, jnp.int32(0xFF))
    nch3 = lax.shift_right_logical(nspk, jnp.int32(8))

    def _issue(ci, buf, sem):
        pltpu.async_copy(edge_hbm.at[pl.ds(ci * CH, CH)], ebuf.at[buf], sem)

    def _wait(ci, buf, sem):
        pltpu.make_async_copy(
            edge_hbm.at[pl.ds(ci * CH, CH)], ebuf.at[buf], sem).wait()

    def _process(buf):
        @plsc.parallel_loop(0, CH, 16, unroll=4)
        def _(e):
            pk = ebuf[buf, pl.ds(e, 16)]
            s16, d16 = _unpack(pk)
            vs = [plsc.load_gather(u_refs[f], [s16]) for f in range(FPT)]
            for f in range(FPT):
                plsc.addupdate_scatter(acc_refs[f], [d16], vs[f])

    def _step(_, carry):
        @pl.loop(0, NP, step=16, unroll=4)
        def _(i):
            for f in range(FPT):
                acc_refs[f][pl.ds(i, 16)] = u_refs[f][pl.ds(i, 16)]

        _issue(0, 0, sem_a)

        @pl.loop(0, NCHUNK, step=2)
        def _(ci):
            _issue(ci + 1, 1, sem_b)
            _wait(ci, 0, sem_a)
            _process(0)

            @pl.when(ci + 2 < NCHUNK)
            def _():
                _issue(ci + 2, 0, sem_a)

            _wait(ci + 1, 1, sem_b)
            _process(1)

        @pl.loop(0, nch3)
        def _(c3):
            pltpu.sync_copy(edge_hbm.at[pl.ds(E_MAIN + c3 * CH, CH)],
                            ebuf.at[0])
            _process(0)

        @pl.loop(0, NP, step=16, unroll=4)
        def _(i):
            dv = dinv_t[pl.ds(i, 16)]
            for f in range(FPT):
                u_refs[f][pl.ds(i, 16)] = acc_refs[f][pl.ds(i, 16)] * dv

        return carry

    lax.fori_loop(0, nsteps, _step, 0)
    for f in range(FPT):
        pltpu.async_copy(u_refs[f], out_hbm.at[f0 + f], sem_m)
    for f in range(FPT):
        pltpu.make_async_copy(u_refs[f], out_hbm.at[f0 + f], sem_m).wait()


@functools.partial(
    pl.kernel,
    out_type=jax.ShapeDtypeStruct((CP, NP), jnp.float32),
    mesh=_mesh,
    compiler_params=_sc_params,
    scratch_types=[
        pltpu.VMEM((NP,), jnp.float32),
        pltpu.VMEM((NP,), jnp.float32),
        pltpu.VMEM((NP,), jnp.float32),
        pltpu.VMEM((2, CH), jnp.int32),
        pltpu.VMEM((16,), jnp.int32),
        pltpu.SemaphoreType.DMA,
        pltpu.SemaphoreType.DMA,
        pltpu.SemaphoreType.DMA,
    ],
)
def _final_kernel(z_hbm, edge_hbm, disq_hbm, ns_hbm, out_hbm,
                  z_t, acc_t, disq_t, ebuf, nsv, sem_a, sem_b, sem_m):
    wid = _wid()

    @pl.when(wid < CP)
    def _():
        pltpu.async_copy(z_hbm.at[wid], z_t, sem_m).wait()
        pltpu.async_copy(disq_hbm, disq_t, sem_m).wait()
        pltpu.async_copy(ns_hbm, nsv, sem_m).wait()
        nch3 = lax.shift_right_logical(jnp.max(nsv[...]), jnp.int32(8))

        @pl.loop(0, NP, step=16, unroll=4)
        def _(i):
            acc_t[pl.ds(i, 16)] = z_t[pl.ds(i, 16)]

        def _issue(ci, buf, sem):
            pltpu.async_copy(edge_hbm.at[pl.ds(ci * CH, CH)], ebuf.at[buf], sem)

        def _wait(ci, buf, sem):
            pltpu.make_async_copy(
                edge_hbm.at[pl.ds(ci * CH, CH)], ebuf.at[buf], sem).wait()

        def _process(buf):
            @plsc.parallel_loop(0, CH, 16, unroll=4)
            def _(e):
                pk = ebuf[buf, pl.ds(e, 16)]
                s16, d16 = _unpack(pk)
                v = plsc.load_gather(z_t, [s16])
                plsc.addupdate_scatter(acc_t, [d16], v)

        _issue(0, 0, sem_a)

        @pl.loop(0, NCHUNK, step=2)
        def _(ci):
            _issue(ci + 1, 1, sem_b)
            _wait(ci, 0, sem_a)
            _process(0)

            @pl.when(ci + 2 < NCHUNK)
            def _():
                _issue(ci + 2, 0, sem_a)

            _wait(ci + 1, 1, sem_b)
            _process(1)

        @pl.loop(0, nch3)
        def _(c3):
            pltpu.sync_copy(edge_hbm.at[pl.ds(E_MAIN + c3 * CH, CH)],
                            ebuf.at[0])
            _process(0)

        @pl.loop(0, NP, step=16, unroll=4)
        def _(i):
            acc_t[pl.ds(i, 16)] = acc_t[pl.ds(i, 16)] * disq_t[pl.ds(i, 16)]

        pltpu.sync_copy(acc_t, out_hbm.at[wid])


def _mm0_body(w_ref, x_ref, b_ref, o_ref):
    o_ref[...] = lax.dot_general(
        w_ref[...], x_ref[...], (((0,), (1,)), ((), ())),
        preferred_element_type=jnp.float32) + b_ref[...]


_mm0 = pl.pallas_call(
    _mm0_body,
    grid=(NP // COLB,),
    in_specs=[
        pl.BlockSpec((D, D), lambda i: (0, 0)),
        pl.BlockSpec((COLB, D), lambda i: (i, 0)),
        pl.BlockSpec((D, 1), lambda i: (0, 0)),
    ],
    out_specs=pl.BlockSpec((D, COLB), lambda i: (0, i)),
    out_shape=jax.ShapeDtypeStruct((D, NP), jnp.float32),
)


def _scale_body(h_ref, hist_ref, u0_ref, dinv_ref, dsq_ref, disq_ref):
    deg = 1.0 + jnp.sum(hist_ref[...], axis=0, keepdims=True)
    pos = deg > 0
    dinv_ref[...] = jnp.where(pos, 1.0 / deg, 0.0)
    sq = jnp.sqrt(deg)
    dsq_ref[...] = sq
    disq = jnp.where(pos, 1.0 / sq, 0.0)
    disq_ref[...] = disq
    u0_ref[...] = h_ref[...] * disq


_scale = pl.pallas_call(
    _scale_body,
    grid=(NP // COLB,),
    in_specs=[
        pl.BlockSpec((D, COLB), lambda i: (0, i)),
        pl.BlockSpec((NTILES, COLB), lambda i: (0, i)),
    ],
    out_specs=[
        pl.BlockSpec((D, COLB), lambda i: (0, i)),
        pl.BlockSpec((1, COLB), lambda i: (0, i)),
        pl.BlockSpec((1, COLB), lambda i: (0, i)),
        pl.BlockSpec((1, COLB), lambda i: (0, i)),
    ],
    out_shape=[
        jax.ShapeDtypeStruct((D, NP), jnp.float32),
        jax.ShapeDtypeStruct((1, NP), jnp.float32),
        jax.ShapeDtypeStruct((1, NP), jnp.float32),
        jax.ShapeDtypeStruct((1, NP), jnp.float32),
    ],
)


def _clf_body(wt_ref, u_ref, dsq_ref, disq_ref, bc_ref, z_ref):
    y = jnp.maximum(u_ref[...], 0.0) * dsq_ref[...]
    z = lax.dot_general(
        wt_ref[...], y, (((1,), (0,)), ((), ())),
        preferred_element_type=jnp.float32)
    z_ref[...] = (z + bc_ref[...]) * disq_ref[...]


_clf = pl.pallas_call(
    _clf_body,
    grid=(NP // COLB,),
    in_specs=[
        pl.BlockSpec((CP, D), lambda i: (0, 0)),
        pl.BlockSpec((D, COLB), lambda i: (0, i)),
        pl.BlockSpec((1, COLB), lambda i: (0, i)),
        pl.BlockSpec((1, COLB), lambda i: (0, i)),
        pl.BlockSpec((CP, 1), lambda i: (0, 0)),
    ],
    out_specs=pl.BlockSpec((CP, COLB), lambda i: (0, i)),
    out_shape=jax.ShapeDtypeStruct((CP, NP), jnp.float32),
)


def kernel(x, edge_index, prop_nums, W0, b0, Wc, bc):
    src = edge_index[0].astype(jnp.int32)
    dst = edge_index[1].astype(jnp.int32)
    pk = jnp.bitwise_or(src, jnp.left_shift(dst, 16))
    cnt = _cnt_kernel(pk)
    base = jnp.cumsum(cnt, axis=0) - cnt
    total = jnp.sum(cnt, axis=0)
    tot_sp = jnp.maximum(total - CAPV, 0)
    s_excl = jnp.cumsum(tot_sp) - tot_sp
    spill_base = s_excl[None, :] + jnp.maximum(base - CAPV, 0)
    pos = _pos_kernel(pk, base, spill_base)
    ovf_hi = jnp.maximum(jnp.max(pos) - (E_MAIN - 1), 0)
    nch3 = (ovf_hi + CH - 1) // CH
    t = jnp.arange(E_TOT, dtype=jnp.int32)
    lpad = t % 16
    kpad = (t // REG) % 16
    spad = N_NODES + lpad
    dpad = N_NODES + ((lpad + kpad) % 16)
    pad_pk = jnp.bitwise_or(spad, jnp.left_shift(dpad, 16))
    epk = pad_pk.at[pos].set(pk)

    x_pad = jnp.pad(x, ((0, NP - N_NODES), (0, 0)))
    b0c = b0.reshape(D, 1)
    wct = jnp.pad(Wc, ((0, 0), (0, CP - C_OUT))).T
    bcp = jnp.pad(bc, (0, CP - C_OUT)).reshape(CP, 1)
    ns_arr = jnp.full((16,), prop_nums + nch3 * 256, jnp.int32)

    hist = _deg_kernel(epk)
    h0t = _mm0(W0, x_pad, b0c)
    u0, dinv, dsq, disq = _scale(h0t, hist)
    u30 = _prop_kernel(u0, epk, dinv.reshape(NP), ns_arr)
    z2 = _clf(wct, u30, dsq, disq, bcp)
    outt = _final_kernel(z2, epk, disq.reshape(NP), ns_arr)
    return outt[:C_OUT, :N_NODES].T


# speedup vs baseline: 1.4430x; 1.0002x over previous
import functools

import jax
import jax.numpy as jnp
from jax import lax
from jax.experimental import pallas as pl
from jax.experimental.pallas import tpu as pltpu
from jax.experimental.pallas import tpu_sc as plsc

N_NODES = 10000
NP = 10240
D = 128
C_OUT = 10
CP = 16
E_RAW = 320000
CAPV = 1344
REG = CAPV * 16
E_BUCK = 256 * CAPV
E_SPARE = 16384
E_MAIN = E_BUCK + E_SPARE
CH = 16384
NCHUNK = E_MAIN // CH
NCH3 = 19
E_TOT = E_MAIN + NCH3 * CH
NTILES = 32
FPT = D // NTILES
ESH = E_TOT // NTILES
COLB = 1024

_mesh = plsc.VectorSubcoreMesh(core_axis_name="c", subcore_axis_name="s")
_sc_params = pltpu.CompilerParams(needs_layout_passes=False)


def _wid():
    return lax.axis_index("c") * 16 + lax.axis_index("s")


def _unpack(pk):
    s16 = lax.bitwise_and(pk, jnp.int32(0xFFFF))
    d16 = lax.shift_right_logical(pk, jnp.int32(16))
    return s16, d16


ESH1 = E_RAW // NTILES


@functools.partial(
    pl.kernel,
    out_type=jax.ShapeDtypeStruct((NTILES, 256), jnp.int32),
    mesh=_mesh,
    compiler_params=_sc_params,
    scratch_types=[
        pltpu.VMEM((256,), jnp.int32),
        pltpu.VMEM((ESH1,), jnp.int32),
        pltpu.SemaphoreType.DMA,
    ],
)
def _cnt_kernel(pk_hbm, cnt_hbm, cnt_t, pkbuf, sem):
    wid = _wid()
    pltpu.async_copy(pk_hbm.at[pl.ds(wid * ESH1, ESH1)], pkbuf, sem).wait()

    @pl.loop(0, 256, step=16)
    def _(i):
        cnt_t[pl.ds(i, 16)] = jnp.zeros((16,), jnp.int32)

    one16 = jnp.ones((16,), jnp.int32)

    @plsc.parallel_loop(0, ESH1, 16, unroll=2)
    def _(e):
        pk16 = pkbuf[pl.ds(e, 16)]
        sl = lax.bitwise_and(pk16, jnp.int32(0xFFFF))
        dl = lax.shift_right_logical(pk16, jnp.int32(16))
        b16 = (lax.bitwise_and(dl - sl, jnp.int32(15)) * 16
               + lax.bitwise_and(sl, jnp.int32(15)))
        plsc.addupdate_scatter(cnt_t, [b16], one16)

    pltpu.sync_copy(cnt_t, cnt_hbm.at[wid])


@functools.partial(
    pl.kernel,
    out_type=jax.ShapeDtypeStruct((E_RAW,), jnp.int32),
    mesh=_mesh,
    compiler_params=_sc_params,
    scratch_types=[
        pltpu.VMEM((256,), jnp.int32),
        pltpu.VMEM((256,), jnp.int32),
        pltpu.VMEM((ESH1,), jnp.int32),
        pltpu.VMEM((ESH1,), jnp.int32),
        pltpu.SemaphoreType.DMA,
    ],
)
def _pos_kernel(pk_hbm, base_hbm, sbase_hbm, pos_hbm,
                cnt_t, scnt_t, pkbuf, posbuf, sem):
    wid = _wid()
    pltpu.async_copy(pk_hbm.at[pl.ds(wid * ESH1, ESH1)], pkbuf, sem).wait()
    pltpu.async_copy(base_hbm.at[wid], cnt_t, sem).wait()
    pltpu.async_copy(sbase_hbm.at[wid], scnt_t, sem).wait()

    @pl.loop(0, ESH1, step=16)
    def _(e):
        pk16 = pkbuf[pl.ds(e, 16)]
        sl = lax.bitwise_and(pk16, jnp.int32(0xFFFF))
        dl = lax.shift_right_logical(pk16, jnp.int32(16))
        b16 = (lax.bitwise_and(dl - sl, jnp.int32(15)) * 16
               + lax.bitwise_and(sl, jnp.int32(15)))
        cum, last = plsc.scan_count(b16)
        old = plsc.load_gather(cnt_t, [b16])
        g = old + cum - 1
        plsc.addupdate_scatter(cnt_t, [b16], cum, mask=last)
        incap = g < CAPV
        posm = (lax.shift_right_logical(b16, jnp.int32(4)) * REG + g * 16
                + lax.bitwise_and(b16, jnp.int32(15)))
        notcap = jnp.logical_not(incap)
        scum, slast = plsc.scan_count(b16, mask=notcap)
        sold = plsc.load_gather(scnt_t, [b16])
        sp = sold + scum - 1
        plsc.addupdate_scatter(scnt_t, [b16], scum,
                               mask=jnp.logical_and(slast, notcap))
        posbuf[pl.ds(e, 16)] = jnp.where(incap, posm, E_BUCK + sp)

    pltpu.sync_copy(posbuf, pos_hbm.at[pl.ds(wid * ESH1, ESH1)])


@functools.partial(
    pl.kernel,
    out_type=jax.ShapeDtypeStruct((NTILES, NP), jnp.float32),
    mesh=_mesh,
    compiler_params=_sc_params,
    scratch_types=[
        pltpu.VMEM((NP,), jnp.float32),
        pltpu.VMEM((ESH,), jnp.int32),
        pltpu.SemaphoreType.DMA,
    ],
)
def _deg_kernel(edge_hbm, hist_hbm, hist_t, ebuf, sem):
    wid = _wid()
    pltpu.async_copy(edge_hbm.at[pl.ds(wid * ESH, ESH)], ebuf, sem).wait()

    @pl.loop(0, NP, step=16, unroll=4)
    def _(i):
        hist_t[pl.ds(i, 16)] = jnp.zeros((16,), jnp.float32)

    one16 = jnp.ones((16,), jnp.float32)

    @plsc.parallel_loop(0, ESH, 16, unroll=4)
    def _(e):
        pk = ebuf[pl.ds(e, 16)]
        _, d16 = _unpack(pk)
        plsc.addupdate_scatter(hist_t, [d16], one16)

    pltpu.sync_copy(hist_t, hist_hbm.at[wid])


@functools.partial(
    pl.kernel,
    out_type=jax.ShapeDtypeStruct((D, NP), jnp.float32),
    mesh=_mesh,
    compiler_params=_sc_params,
    scratch_types=[
        [pltpu.VMEM((NP,), jnp.float32)] * FPT,
        [pltpu.VMEM((NP,), jnp.float32)] * FPT,
        pltpu.VMEM((NP,), jnp.float32),
        pltpu.VMEM((2, CH), jnp.int32),
        pltpu.VMEM((16,), jnp.int32),
        pltpu.SemaphoreType.DMA,
        pltpu.SemaphoreType.DMA,
        pltpu.SemaphoreType.DMA,
    ],
)
def _prop_kernel(u0_hbm, edge_hbm, dinv_hbm, ns_hbm, out_hbm,
                 u_refs, acc_refs, dinv_t, ebuf, nsv, sem_a, sem_b, sem_m):
    wid = _wid()
    f0 = wid * FPT
    for f in range(FPT):
        pltpu.async_copy(u0_hbm.at[f0 + f], u_refs[f], sem_m)
    pltpu.async_copy(dinv_hbm, dinv_t, sem_m)
    pltpu.async_copy(ns_hbm, nsv, sem_m)
    for f in range(FPT):
        pltpu.make_async_copy(u0_hbm.at[f0 + f], u_refs[f], sem_m).wait()
    pltpu.make_async_copy(dinv_hbm, dinv_t, sem_m).wait()
    pltpu.make_async_copy(ns_hbm, nsv, sem_m).wait()
    nspk = jnp.max(nsv[...])
    nsteps = lax.bitwise_and(nspk, jnp.int32(0xFF))
    nch3 = lax.shift_right_logical(nspk, jnp.int32(8))

    def _issue(ci, buf, sem):
        pltpu.async_copy(edge_hbm.at[pl.ds(ci * CH, CH)], ebuf.at[buf], sem)

    def _wait(ci, buf, sem):
        pltpu.make_async_copy(
            edge_hbm.at[pl.ds(ci * CH, CH)], ebuf.at[buf], sem).wait()

    def _process(buf):
        @plsc.parallel_loop(0, CH, 16, unroll=4)
        def _(e):
            pk = ebuf[buf, pl.ds(e, 16)]
            s16, d16 = _unpack(pk)
            vs = [plsc.load_gather(u_refs[f], [s16]) for f in range(FPT)]
            for f in range(FPT):
                plsc.addupdate_scatter(acc_refs[f], [d16], vs[f])

    def _step(_, carry):
        @pl.loop(0, NP, step=16, unroll=4)
        def _(i):
            for f in range(FPT):
                acc_refs[f][pl.ds(i, 16)] = u_refs[f][pl.ds(i, 16)]

        _issue(0, 0, sem_a)

        @pl.loop(0, NCHUNK, step=2)
        def _(ci):
            _issue(ci + 1, 1, sem_b)
            _wait(ci, 0, sem_a)
            _process(0)

            @pl.when(ci + 2 < NCHUNK)
            def _():
                _issue(ci + 2, 0, sem_a)

            _wait(ci + 1, 1, sem_b)
            _process(1)

        @pl.loop(0, nch3)
        def _(c3):
            pltpu.sync_copy(edge_hbm.at[pl.ds(E_MAIN + c3 * CH, CH)],
                            ebuf.at[0])
            _process(0)

        @pl.loop(0, NP, step=16, unroll=4)
        def _(i):
            dv = dinv_t[pl.ds(i, 16)]
            for f in range(FPT):
                u_refs[f][pl.ds(i, 16)] = acc_refs[f][pl.ds(i, 16)] * dv

        return carry

    lax.fori_loop(0, nsteps, _step, 0)
    for f in range(FPT):
        pltpu.async_copy(u_refs[f], out_hbm.at[f0 + f], sem_m)
    for f in range(FPT):
        pltpu.make_async_copy(u_refs[f], out_hbm.at[f0 + f], sem_m).wait()


@functools.partial(
    pl.kernel,
    out_type=jax.ShapeDtypeStruct((CP, NP), jnp.float32),
    mesh=_mesh,
    compiler_params=_sc_params,
    scratch_types=[
        pltpu.VMEM((NP,), jnp.float32),
        pltpu.VMEM((NP,), jnp.float32),
        pltpu.VMEM((NP,), jnp.float32),
        pltpu.VMEM((2, CH), jnp.int32),
        pltpu.VMEM((16,), jnp.int32),
        pltpu.SemaphoreType.DMA,
        pltpu.SemaphoreType.DMA,
        pltpu.SemaphoreType.DMA,
    ],
)
def _final_kernel(z_hbm, edge_hbm, disq_hbm, ns_hbm, out_hbm,
                  z_t, acc_t, disq_t, ebuf, nsv, sem_a, sem_b, sem_m):
    wid = _wid()

    @pl.when(wid < CP)
    def _():
        pltpu.async_copy(z_hbm.at[wid], z_t, sem_m).wait()
        pltpu.async_copy(disq_hbm, disq_t, sem_m).wait()
        pltpu.async_copy(ns_hbm, nsv, sem_m).wait()
        nch3 = lax.shift_right_logical(jnp.max(nsv[...]), jnp.int32(8))

        @pl.loop(0, NP, step=16, unroll=4)
        def _(i):
            acc_t[pl.ds(i, 16)] = z_t[pl.ds(i, 16)]

        def _issue(ci, buf, sem):
            pltpu.async_copy(edge_hbm.at[pl.ds(ci * CH, CH)], ebuf.at[buf], sem)

        def _wait(ci, buf, sem):
            pltpu.make_async_copy(
                edge_hbm.at[pl.ds(ci * CH, CH)], ebuf.at[buf], sem).wait()

        def _process(buf):
            @plsc.parallel_loop(0, CH, 16, unroll=4)
            def _(e):
                pk = ebuf[buf, pl.ds(e, 16)]
                s16, d16 = _unpack(pk)
                v = plsc.load_gather(z_t, [s16])
                plsc.addupdate_scatter(acc_t, [d16], v)

        _issue(0, 0, sem_a)

        @pl.loop(0, NCHUNK, step=2)
        def _(ci):
            _issue(ci + 1, 1, sem_b)
            _wait(ci, 0, sem_a)
            _process(0)

            @pl.when(ci + 2 < NCHUNK)
            def _():
                _issue(ci + 2, 0, sem_a)

            _wait(ci + 1, 1, sem_b)
            _process(1)

        @pl.loop(0, nch3)
        def _(c3):
            pltpu.sync_copy(edge_hbm.at[pl.ds(E_MAIN + c3 * CH, CH)],
                            ebuf.at[0])
            _process(0)

        @pl.loop(0, NP, step=16, unroll=4)
        def _(i):
            acc_t[pl.ds(i, 16)] = acc_t[pl.ds(i, 16)] * disq_t[pl.ds(i, 16)]

        pltpu.sync_copy(acc_t, out_hbm.at[wid])


def _mm0_body(w_ref, x_ref, b_ref, o_ref):
    o_ref[...] = lax.dot_general(
        w_ref[...], x_ref[...], (((0,), (1,)), ((), ())),
        preferred_element_type=jnp.float32) + b_ref[...]


_mm0 = pl.pallas_call(
    _mm0_body,
    grid=(NP // COLB,),
    in_specs=[
        pl.BlockSpec((D, D), lambda i: (0, 0)),
        pl.BlockSpec((COLB, D), lambda i: (i, 0)),
        pl.BlockSpec((D, 1), lambda i: (0, 0)),
    ],
    out_specs=pl.BlockSpec((D, COLB), lambda i: (0, i)),
    out_shape=jax.ShapeDtypeStruct((D, NP), jnp.float32),
)


def _scale_body(h_ref, hist_ref, u0_ref, dinv_ref, dsq_ref, disq_ref):
    deg = 1.0 + jnp.sum(hist_ref[...], axis=0, keepdims=True)
    pos = deg > 0
    dinv_ref[...] = jnp.where(pos, 1.0 / deg, 0.0)
    sq = jnp.sqrt(deg)
    dsq_ref[...] = sq
    disq = jnp.where(pos, 1.0 / sq, 0.0)
    disq_ref[...] = disq
    u0_ref[...] = h_ref[...] * disq


_scale = pl.pallas_call(
    _scale_body,
    grid=(NP // COLB,),
    in_specs=[
        pl.BlockSpec((D, COLB), lambda i: (0, i)),
        pl.BlockSpec((NTILES, COLB), lambda i: (0, i)),
    ],
    out_specs=[
        pl.BlockSpec((D, COLB), lambda i: (0, i)),
        pl.BlockSpec((1, COLB), lambda i: (0, i)),
        pl.BlockSpec((1, COLB), lambda i: (0, i)),
        pl.BlockSpec((1, COLB), lambda i: (0, i)),
    ],
    out_shape=[
        jax.ShapeDtypeStruct((D, NP), jnp.float32),
        jax.ShapeDtypeStruct((1, NP), jnp.float32),
        jax.ShapeDtypeStruct((1, NP), jnp.float32),
        jax.ShapeDtypeStruct((1, NP), jnp.float32),
    ],
)


def _clf_body(wt_ref, u_ref, dsq_ref, disq_ref, bc_ref, z_ref):
    y = jnp.maximum(u_ref[...], 0.0) * dsq_ref[...]
    z = lax.dot_general(
        wt_ref[...], y, (((1,), (0,)), ((), ())),
        preferred_element_type=jnp.float32)
    z_ref[...] = (z + bc_ref[...]) * disq_ref[...]


_clf = pl.pallas_call(
    _clf_body,
    grid=(NP // COLB,),
    in_specs=[
        pl.BlockSpec((CP, D), lambda i: (0, 0)),
        pl.BlockSpec((D, COLB), lambda i: (0, i)),
        pl.BlockSpec((1, COLB), lambda i: (0, i)),
        pl.BlockSpec((1, COLB), lambda i: (0, i)),
        pl.BlockSpec((CP, 1), lambda i: (0, 0)),
    ],
    out_specs=pl.BlockSpec((CP, COLB), lambda i: (0, i)),
    out_shape=jax.ShapeDtypeStruct((CP, NP), jnp.float32),
)


def kernel(x, edge_index, prop_nums, W0, b0, Wc, bc):
    src = edge_index[0].astype(jnp.int32)
    dst = edge_index[1].astype(jnp.int32)
    pk = jnp.bitwise_or(src, jnp.left_shift(dst, 16))
    cnt = _cnt_kernel(pk)
    base = jnp.cumsum(cnt, axis=0) - cnt
    total = jnp.sum(cnt, axis=0)
    tot_sp = jnp.maximum(total - CAPV, 0)
    s_excl = jnp.cumsum(tot_sp) - tot_sp
    spill_base = s_excl[None, :] + jnp.maximum(base - CAPV, 0)
    pos = _pos_kernel(pk, base, spill_base)
    ovf_hi = jnp.maximum(jnp.max(pos) - (E_MAIN - 1), 0)
    nch3 = (ovf_hi + CH - 1) // CH
    t = jnp.arange(E_TOT, dtype=jnp.int32)
    lpad = t % 16
    kpad = (t // REG) % 16
    spad = N_NODES + lpad
    dpad = N_NODES + ((lpad + kpad) % 16)
    pad_pk = jnp.bitwise_or(spad, jnp.left_shift(dpad, 16))
    epk = pad_pk.at[pos].set(pk, mode='promise_in_bounds', unique_indices=True)

    x_pad = jnp.pad(x, ((0, NP - N_NODES), (0, 0)))
    b0c = b0.reshape(D, 1)
    wct = jnp.pad(Wc, ((0, 0), (0, CP - C_OUT))).T
    bcp = jnp.pad(bc, (0, CP - C_OUT)).reshape(CP, 1)
    ns_arr = jnp.full((16,), prop_nums + nch3 * 256, jnp.int32)

    hist = _deg_kernel(epk)
    h0t = _mm0(W0, x_pad, b0c)
    u0, dinv, dsq, disq = _scale(h0t, hist)
    u30 = _prop_kernel(u0, epk, dinv.reshape(NP), ns_arr)
    z2 = _clf(wct, u30, dsq, disq, bcp)
    outt = _final_kernel(z2, epk, disq.reshape(NP), ns_arr)
    return outt[:C_OUT, :N_NODES].T


# speedup vs baseline: 1.6678x; 1.1558x over previous
import functools

import jax
import jax.numpy as jnp
from jax import lax
from jax.experimental import pallas as pl
from jax.experimental.pallas import tpu as pltpu
from jax.experimental.pallas import tpu_sc as plsc

N_NODES = 10000
NP = 10240
D = 128
C_OUT = 10
CP = 16
E_RAW = 320000
CAPV = 1344
REG = CAPV * 16
E_BUCK = 256 * CAPV
E_SPARE = 16384
E_MAIN = E_BUCK + E_SPARE
CH = 16384
NCHUNK = E_MAIN // CH
NCH3 = 19
E_TOT = E_MAIN + NCH3 * CH
NTILES = 32
FPT = D // NTILES
ESH = E_TOT // NTILES
COLB = 1024

_mesh = plsc.VectorSubcoreMesh(core_axis_name="c", subcore_axis_name="s")
_sc_params = pltpu.CompilerParams(needs_layout_passes=False)


def _wid():
    return lax.axis_index("c") * 16 + lax.axis_index("s")


def _unpack(pk):
    s16 = lax.bitwise_and(pk, jnp.int32(0xFFFF))
    d16 = lax.shift_right_logical(pk, jnp.int32(16))
    return s16, d16


ESH1 = E_RAW // NTILES


@functools.partial(
    pl.kernel,
    out_type=jax.ShapeDtypeStruct((NTILES, 256), jnp.int32),
    mesh=_mesh,
    compiler_params=_sc_params,
    scratch_types=[
        pltpu.VMEM((256,), jnp.int32),
        pltpu.VMEM((ESH1,), jnp.int32),
        pltpu.SemaphoreType.DMA,
    ],
)
def _cnt_kernel(pk_hbm, cnt_hbm, cnt_t, pkbuf, sem):
    wid = _wid()
    pltpu.async_copy(pk_hbm.at[pl.ds(wid * ESH1, ESH1)], pkbuf, sem).wait()

    @pl.loop(0, 256, step=16)
    def _(i):
        cnt_t[pl.ds(i, 16)] = jnp.zeros((16,), jnp.int32)

    one16 = jnp.ones((16,), jnp.int32)

    @plsc.parallel_loop(0, ESH1, 16, unroll=2)
    def _(e):
        pk16 = pkbuf[pl.ds(e, 16)]
        sl = lax.bitwise_and(pk16, jnp.int32(0xFFFF))
        dl = lax.shift_right_logical(pk16, jnp.int32(16))
        b16 = (lax.bitwise_and(dl - sl, jnp.int32(15)) * 16
               + lax.bitwise_and(sl, jnp.int32(15)))
        plsc.addupdate_scatter(cnt_t, [b16], one16)

    pltpu.sync_copy(cnt_t, cnt_hbm.at[wid])


@functools.partial(
    pl.kernel,
    out_type=jax.ShapeDtypeStruct((E_TOT,), jnp.int32),
    mesh=_mesh,
    compiler_params=_sc_params,
    scratch_types=[
        pltpu.VMEM((256,), jnp.int32),
        pltpu.VMEM((256,), jnp.int32),
        pltpu.VMEM((ESH1,), jnp.int32),
        pltpu.VMEM((ESH1,), jnp.int32),
        pltpu.SemaphoreType.DMA,
    ],
)
def _pos_kernel(pk_hbm, base_hbm, sbase_hbm, epk_hbm,
                cnt_t, scnt_t, pkbuf, posbuf, sem):
    wid = _wid()
    pltpu.async_copy(pk_hbm.at[pl.ds(wid * ESH1, ESH1)], pkbuf, sem).wait()
    pltpu.async_copy(base_hbm.at[wid], cnt_t, sem).wait()
    pltpu.async_copy(sbase_hbm.at[wid], scnt_t, sem).wait()

    @pl.loop(0, ESH1, step=16)
    def _(e):
        pk16 = pkbuf[pl.ds(e, 16)]
        sl = lax.bitwise_and(pk16, jnp.int32(0xFFFF))
        dl = lax.shift_right_logical(pk16, jnp.int32(16))
        b16 = (lax.bitwise_and(dl - sl, jnp.int32(15)) * 16
               + lax.bitwise_and(sl, jnp.int32(15)))
        cum, last = plsc.scan_count(b16)
        old = plsc.load_gather(cnt_t, [b16])
        g = old + cum - 1
        plsc.addupdate_scatter(cnt_t, [b16], cum, mask=last)
        incap = g < CAPV
        posm = (lax.shift_right_logical(b16, jnp.int32(4)) * REG + g * 16
                + lax.bitwise_and(b16, jnp.int32(15)))
        notcap = jnp.logical_not(incap)
        scum, slast = plsc.scan_count(b16, mask=notcap)
        sold = plsc.load_gather(scnt_t, [b16])
        sp = sold + scum - 1
        plsc.addupdate_scatter(scnt_t, [b16], scum,
                               mask=jnp.logical_and(slast, notcap))
        posbuf[pl.ds(e, 16)] = jnp.where(incap, posm, E_BUCK + sp)

    pltpu.async_copy(pkbuf, epk_hbm.at[posbuf], sem).wait()


@functools.partial(
    pl.kernel,
    out_type=jax.ShapeDtypeStruct((NTILES, NP), jnp.float32),
    mesh=_mesh,
    compiler_params=_sc_params,
    scratch_types=[
        pltpu.VMEM((NP,), jnp.float32),
        pltpu.VMEM((ESH,), jnp.int32),
        pltpu.SemaphoreType.DMA,
    ],
)
def _deg_kernel(edge_hbm, hist_hbm, hist_t, ebuf, sem):
    wid = _wid()
    pltpu.async_copy(edge_hbm.at[pl.ds(wid * ESH, ESH)], ebuf, sem).wait()

    @pl.loop(0, NP, step=16, unroll=4)
    def _(i):
        hist_t[pl.ds(i, 16)] = jnp.zeros((16,), jnp.float32)

    one16 = jnp.ones((16,), jnp.float32)

    @plsc.parallel_loop(0, ESH, 16, unroll=4)
    def _(e):
        pk = ebuf[pl.ds(e, 16)]
        _, d16 = _unpack(pk)
        plsc.addupdate_scatter(hist_t, [d16], one16)

    pltpu.sync_copy(hist_t, hist_hbm.at[wid])


@functools.partial(
    pl.kernel,
    out_type=jax.ShapeDtypeStruct((D, NP), jnp.float32),
    mesh=_mesh,
    compiler_params=_sc_params,
    scratch_types=[
        [pltpu.VMEM((NP,), jnp.float32)] * FPT,
        [pltpu.VMEM((NP,), jnp.float32)] * FPT,
        pltpu.VMEM((NP,), jnp.float32),
        pltpu.VMEM((2, CH), jnp.int32),
        pltpu.VMEM((16,), jnp.int32),
        pltpu.SemaphoreType.DMA,
        pltpu.SemaphoreType.DMA,
        pltpu.SemaphoreType.DMA,
    ],
)
def _prop_kernel(u0_hbm, edge_hbm, dinv_hbm, ns_hbm, out_hbm,
                 u_refs, acc_refs, dinv_t, ebuf, nsv, sem_a, sem_b, sem_m):
    wid = _wid()
    f0 = wid * FPT
    for f in range(FPT):
        pltpu.async_copy(u0_hbm.at[f0 + f], u_refs[f], sem_m)
    pltpu.async_copy(dinv_hbm, dinv_t, sem_m)
    pltpu.async_copy(ns_hbm, nsv, sem_m)
    for f in range(FPT):
        pltpu.make_async_copy(u0_hbm.at[f0 + f], u_refs[f], sem_m).wait()
    pltpu.make_async_copy(dinv_hbm, dinv_t, sem_m).wait()
    pltpu.make_async_copy(ns_hbm, nsv, sem_m).wait()
    nspk = jnp.max(nsv[...])
    nsteps = lax.bitwise_and(nspk, jnp.int32(0xFF))
    nch3 = lax.shift_right_logical(nspk, jnp.int32(8))

    def _issue(ci, buf, sem):
        pltpu.async_copy(edge_hbm.at[pl.ds(ci * CH, CH)], ebuf.at[buf], sem)

    def _wait(ci, buf, sem):
        pltpu.make_async_copy(
            edge_hbm.at[pl.ds(ci * CH, CH)], ebuf.at[buf], sem).wait()

    def _process(buf):
        @plsc.parallel_loop(0, CH, 16, unroll=4)
        def _(e):
            pk = ebuf[buf, pl.ds(e, 16)]
            s16, d16 = _unpack(pk)
            vs = [plsc.load_gather(u_refs[f], [s16]) for f in range(FPT)]
            for f in range(FPT):
                plsc.addupdate_scatter(acc_refs[f], [d16], vs[f])

    def _step(_, carry):
        @pl.loop(0, NP, step=16, unroll=4)
        def _(i):
            for f in range(FPT):
                acc_refs[f][pl.ds(i, 16)] = u_refs[f][pl.ds(i, 16)]

        _issue(0, 0, sem_a)

        @pl.loop(0, NCHUNK, step=2)
        def _(ci):
            _issue(ci + 1, 1, sem_b)
            _wait(ci, 0, sem_a)
            _process(0)

            @pl.when(ci + 2 < NCHUNK)
            def _():
                _issue(ci + 2, 0, sem_a)

            _wait(ci + 1, 1, sem_b)
            _process(1)

        @pl.loop(0, nch3)
        def _(c3):
            pltpu.sync_copy(edge_hbm.at[pl.ds(E_MAIN + c3 * CH, CH)],
                            ebuf.at[0])
            _process(0)

        @pl.loop(0, NP, step=16, unroll=4)
        def _(i):
            dv = dinv_t[pl.ds(i, 16)]
            for f in range(FPT):
                u_refs[f][pl.ds(i, 16)] = acc_refs[f][pl.ds(i, 16)] * dv

        return carry

    lax.fori_loop(0, nsteps, _step, 0)
    for f in range(FPT):
        pltpu.async_copy(u_refs[f], out_hbm.at[f0 + f], sem_m)
    for f in range(FPT):
        pltpu.make_async_copy(u_refs[f], out_hbm.at[f0 + f], sem_m).wait()


@functools.partial(
    pl.kernel,
    out_type=jax.ShapeDtypeStruct((CP, NP), jnp.float32),
    mesh=_mesh,
    compiler_params=_sc_params,
    scratch_types=[
        pltpu.VMEM((NP,), jnp.float32),
        pltpu.VMEM((NP,), jnp.float32),
        pltpu.VMEM((NP,), jnp.float32),
        pltpu.VMEM((2, CH), jnp.int32),
        pltpu.VMEM((16,), jnp.int32),
        pltpu.SemaphoreType.DMA,
        pltpu.SemaphoreType.DMA,
        pltpu.SemaphoreType.DMA,
    ],
)
def _final_kernel(z_hbm, edge_hbm, disq_hbm, ns_hbm, out_hbm,
                  z_t, acc_t, disq_t, ebuf, nsv, sem_a, sem_b, sem_m):
    wid = _wid()

    @pl.when(wid < CP)
    def _():
        pltpu.async_copy(z_hbm.at[wid], z_t, sem_m).wait()
        pltpu.async_copy(disq_hbm, disq_t, sem_m).wait()
        pltpu.async_copy(ns_hbm, nsv, sem_m).wait()
        nch3 = lax.shift_right_logical(jnp.max(nsv[...]), jnp.int32(8))

        @pl.loop(0, NP, step=16, unroll=4)
        def _(i):
            acc_t[pl.ds(i, 16)] = z_t[pl.ds(i, 16)]

        def _issue(ci, buf, sem):
            pltpu.async_copy(edge_hbm.at[pl.ds(ci * CH, CH)], ebuf.at[buf], sem)

        def _wait(ci, buf, sem):
            pltpu.make_async_copy(
                edge_hbm.at[pl.ds(ci * CH, CH)], ebuf.at[buf], sem).wait()

        def _process(buf):
            @plsc.parallel_loop(0, CH, 16, unroll=4)
            def _(e):
                pk = ebuf[buf, pl.ds(e, 16)]
                s16, d16 = _unpack(pk)
                v = plsc.load_gather(z_t, [s16])
                plsc.addupdate_scatter(acc_t, [d16], v)

        _issue(0, 0, sem_a)

        @pl.loop(0, NCHUNK, step=2)
        def _(ci):
            _issue(ci + 1, 1, sem_b)
            _wait(ci, 0, sem_a)
            _process(0)

            @pl.when(ci + 2 < NCHUNK)
            def _():
                _issue(ci + 2, 0, sem_a)

            _wait(ci + 1, 1, sem_b)
            _process(1)

        @pl.loop(0, nch3)
        def _(c3):
            pltpu.sync_copy(edge_hbm.at[pl.ds(E_MAIN + c3 * CH, CH)],
                            ebuf.at[0])
            _process(0)

        @pl.loop(0, NP, step=16, unroll=4)
        def _(i):
            acc_t[pl.ds(i, 16)] = acc_t[pl.ds(i, 16)] * disq_t[pl.ds(i, 16)]

        pltpu.sync_copy(acc_t, out_hbm.at[wid])


def _mm0_body(w_ref, x_ref, b_ref, o_ref):
    o_ref[...] = lax.dot_general(
        w_ref[...], x_ref[...], (((0,), (1,)), ((), ())),
        preferred_element_type=jnp.float32) + b_ref[...]


_mm0 = pl.pallas_call(
    _mm0_body,
    grid=(NP // COLB,),
    in_specs=[
        pl.BlockSpec((D, D), lambda i: (0, 0)),
        pl.BlockSpec((COLB, D), lambda i: (i, 0)),
        pl.BlockSpec((D, 1), lambda i: (0, 0)),
    ],
    out_specs=pl.BlockSpec((D, COLB), lambda i: (0, i)),
    out_shape=jax.ShapeDtypeStruct((D, NP), jnp.float32),
)


def _scale_body(h_ref, hist_ref, u0_ref, dinv_ref, dsq_ref, disq_ref):
    deg = 1.0 + jnp.sum(hist_ref[...], axis=0, keepdims=True)
    pos = deg > 0
    dinv_ref[...] = jnp.where(pos, 1.0 / deg, 0.0)
    sq = jnp.sqrt(deg)
    dsq_ref[...] = sq
    disq = jnp.where(pos, 1.0 / sq, 0.0)
    disq_ref[...] = disq
    u0_ref[...] = h_ref[...] * disq


_scale = pl.pallas_call(
    _scale_body,
    grid=(NP // COLB,),
    in_specs=[
        pl.BlockSpec((D, COLB), lambda i: (0, i)),
        pl.BlockSpec((NTILES, COLB), lambda i: (0, i)),
    ],
    out_specs=[
        pl.BlockSpec((D, COLB), lambda i: (0, i)),
        pl.BlockSpec((1, COLB), lambda i: (0, i)),
        pl.BlockSpec((1, COLB), lambda i: (0, i)),
        pl.BlockSpec((1, COLB), lambda i: (0, i)),
    ],
    out_shape=[
        jax.ShapeDtypeStruct((D, NP), jnp.float32),
        jax.ShapeDtypeStruct((1, NP), jnp.float32),
        jax.ShapeDtypeStruct((1, NP), jnp.float32),
        jax.ShapeDtypeStruct((1, NP), jnp.float32),
    ],
)


def _clf_body(wt_ref, u_ref, dsq_ref, disq_ref, bc_ref, z_ref):
    y = jnp.maximum(u_ref[...], 0.0) * dsq_ref[...]
    z = lax.dot_general(
        wt_ref[...], y, (((1,), (0,)), ((), ())),
        preferred_element_type=jnp.float32)
    z_ref[...] = (z + bc_ref[...]) * disq_ref[...]


_clf = pl.pallas_call(
    _clf_body,
    grid=(NP // COLB,),
    in_specs=[
        pl.BlockSpec((CP, D), lambda i: (0, 0)),
        pl.BlockSpec((D, COLB), lambda i: (0, i)),
        pl.BlockSpec((1, COLB), lambda i: (0, i)),
        pl.BlockSpec((1, COLB), lambda i: (0, i)),
        pl.BlockSpec((CP, 1), lambda i: (0, 0)),
    ],
    out_specs=pl.BlockSpec((CP, COLB), lambda i: (0, i)),
    out_shape=jax.ShapeDtypeStruct((CP, NP), jnp.float32),
)


def kernel(x, edge_index, prop_nums, W0, b0, Wc, bc):
    src = edge_index[0].astype(jnp.int32)
    dst = edge_index[1].astype(jnp.int32)
    pk = jnp.bitwise_or(src, jnp.left_shift(dst, 16))
    cnt = _cnt_kernel(pk)
    base = jnp.cumsum(cnt, axis=0) - cnt
    total = jnp.sum(cnt, axis=0)
    tot_sp = jnp.maximum(total - CAPV, 0)
    s_excl = jnp.cumsum(tot_sp) - tot_sp
    spill_base = s_excl[None, :] + jnp.maximum(base - CAPV, 0)
    epk_sc = _pos_kernel(pk, base, spill_base)
    tot2 = jnp.minimum(total, CAPV)
    n_spill = E_RAW - jnp.sum(tot2)
    n_ovf = jnp.maximum(n_spill - E_SPARE, 0)
    nch3 = (n_ovf + CH - 1) // CH
    realb = (jnp.arange(CAPV, dtype=jnp.int32)[None, :, None]
             < tot2.reshape(16, 16)[:, None, :]).reshape(E_BUCK)
    real_sp = jnp.arange(E_TOT - E_BUCK, dtype=jnp.int32) < n_spill
    real_full = jnp.concatenate([realb, real_sp])
    t = jnp.arange(E_TOT, dtype=jnp.int32)
    lpad = t % 16
    kpad = (t // REG) % 16
    spad = N_NODES + lpad
    dpad = N_NODES + ((lpad + kpad) % 16)
    pad_pk = jnp.bitwise_or(spad, jnp.left_shift(dpad, 16))
    epk = jnp.where(real_full, epk_sc, pad_pk)

    x_pad = jnp.pad(x, ((0, NP - N_NODES), (0, 0)))
    b0c = b0.reshape(D, 1)
    wct = jnp.pad(Wc, ((0, 0), (0, CP - C_OUT))).T
    bcp = jnp.pad(bc, (0, CP - C_OUT)).reshape(CP, 1)
    ns_arr = jnp.full((16,), prop_nums + nch3 * 256, jnp.int32)

    hist = _deg_kernel(epk)
    h0t = _mm0(W0, x_pad, b0c)
    u0, dinv, dsq, disq = _scale(h0t, hist)
    u30 = _prop_kernel(u0, epk, dinv.reshape(NP), ns_arr)
    z2 = _clf(wct, u30, dsq, disq, bcp)
    outt = _final_kernel(z2, epk, disq.reshape(NP), ns_arr)
    return outt[:C_OUT, :N_NODES].T


# speedup vs baseline: 1.7302x; 1.0374x over previous
import functools

import jax
import jax.numpy as jnp
from jax import lax
from jax.experimental import pallas as pl
from jax.experimental.pallas import tpu as pltpu
from jax.experimental.pallas import tpu_sc as plsc

N_NODES = 10000
NP = 10240
D = 128
C_OUT = 10
CP = 16
E_RAW = 320000
CAPV = 1344
REG = CAPV * 16
E_BUCK = 256 * CAPV
E_SPARE = 4096
E_MAIN = E_BUCK + E_SPARE
CH = 17408
NCHUNK = E_MAIN // CH
NCH3 = 19
E_TOT = E_MAIN + NCH3 * CH
NTILES = 32
FPT = D // NTILES
ESH = E_TOT // NTILES
COLB = 1024

_mesh = plsc.VectorSubcoreMesh(core_axis_name="c", subcore_axis_name="s")
_sc_params = pltpu.CompilerParams(needs_layout_passes=False)


def _wid():
    return lax.axis_index("c") * 16 + lax.axis_index("s")


def _unpack(pk):
    s16 = lax.bitwise_and(pk, jnp.int32(0xFFFF))
    d16 = lax.shift_right_logical(pk, jnp.int32(16))
    return s16, d16


ESH1 = E_RAW // NTILES


@functools.partial(
    pl.kernel,
    out_type=jax.ShapeDtypeStruct((NTILES, 256), jnp.int32),
    mesh=_mesh,
    compiler_params=_sc_params,
    scratch_types=[
        pltpu.VMEM((256,), jnp.int32),
        pltpu.VMEM((ESH1,), jnp.int32),
        pltpu.SemaphoreType.DMA,
    ],
)
def _cnt_kernel(pk_hbm, cnt_hbm, cnt_t, pkbuf, sem):
    wid = _wid()
    pltpu.async_copy(pk_hbm.at[pl.ds(wid * ESH1, ESH1)], pkbuf, sem).wait()

    @pl.loop(0, 256, step=16)
    def _(i):
        cnt_t[pl.ds(i, 16)] = jnp.zeros((16,), jnp.int32)

    one16 = jnp.ones((16,), jnp.int32)

    @plsc.parallel_loop(0, ESH1, 16, unroll=2)
    def _(e):
        pk16 = pkbuf[pl.ds(e, 16)]
        sl = lax.bitwise_and(pk16, jnp.int32(0xFFFF))
        dl = lax.shift_right_logical(pk16, jnp.int32(16))
        b16 = (lax.bitwise_and(dl - sl, jnp.int32(15)) * 16
               + lax.bitwise_and(sl, jnp.int32(15)))
        plsc.addupdate_scatter(cnt_t, [b16], one16)

    pltpu.sync_copy(cnt_t, cnt_hbm.at[wid])


@functools.partial(
    pl.kernel,
    out_type=jax.ShapeDtypeStruct((E_TOT,), jnp.int32),
    mesh=_mesh,
    compiler_params=_sc_params,
    scratch_types=[
        pltpu.VMEM((256,), jnp.int32),
        pltpu.VMEM((256,), jnp.int32),
        pltpu.VMEM((ESH1,), jnp.int32),
        pltpu.VMEM((ESH1,), jnp.int32),
        pltpu.SemaphoreType.DMA,
    ],
)
def _pos_kernel(pk_hbm, base_hbm, sbase_hbm, epk_hbm,
                cnt_t, scnt_t, pkbuf, posbuf, sem):
    wid = _wid()
    pltpu.async_copy(pk_hbm.at[pl.ds(wid * ESH1, ESH1)], pkbuf, sem).wait()
    pltpu.async_copy(base_hbm.at[wid], cnt_t, sem).wait()
    pltpu.async_copy(sbase_hbm.at[wid], scnt_t, sem).wait()

    @pl.loop(0, ESH1, step=16)
    def _(e):
        pk16 = pkbuf[pl.ds(e, 16)]
        sl = lax.bitwise_and(pk16, jnp.int32(0xFFFF))
        dl = lax.shift_right_logical(pk16, jnp.int32(16))
        b16 = (lax.bitwise_and(dl - sl, jnp.int32(15)) * 16
               + lax.bitwise_and(sl, jnp.int32(15)))
        cum, last = plsc.scan_count(b16)
        old = plsc.load_gather(cnt_t, [b16])
        g = old + cum - 1
        plsc.addupdate_scatter(cnt_t, [b16], cum, mask=last)
        incap = g < CAPV
        posm = (lax.shift_right_logical(b16, jnp.int32(4)) * REG + g * 16
                + lax.bitwise_and(b16, jnp.int32(15)))
        notcap = jnp.logical_not(incap)
        scum, slast = plsc.scan_count(b16, mask=notcap)
        sold = plsc.load_gather(scnt_t, [b16])
        sp = sold + scum - 1
        plsc.addupdate_scatter(scnt_t, [b16], scum,
                               mask=jnp.logical_and(slast, notcap))
        posbuf[pl.ds(e, 16)] = jnp.where(incap, posm, E_BUCK + sp)

    pltpu.async_copy(pkbuf, epk_hbm.at[posbuf], sem).wait()


@functools.partial(
    pl.kernel,
    out_type=jax.ShapeDtypeStruct((NTILES, NP), jnp.float32),
    mesh=_mesh,
    compiler_params=_sc_params,
    scratch_types=[
        pltpu.VMEM((NP,), jnp.float32),
        pltpu.VMEM((ESH,), jnp.int32),
        pltpu.SemaphoreType.DMA,
    ],
)
def _deg_kernel(edge_hbm, hist_hbm, hist_t, ebuf, sem):
    wid = _wid()
    pltpu.async_copy(edge_hbm.at[pl.ds(wid * ESH, ESH)], ebuf, sem).wait()

    @pl.loop(0, NP, step=16, unroll=4)
    def _(i):
        hist_t[pl.ds(i, 16)] = jnp.zeros((16,), jnp.float32)

    one16 = jnp.ones((16,), jnp.float32)

    @plsc.parallel_loop(0, ESH, 16, unroll=4)
    def _(e):
        pk = ebuf[pl.ds(e, 16)]
        _, d16 = _unpack(pk)
        plsc.addupdate_scatter(hist_t, [d16], one16)

    pltpu.sync_copy(hist_t, hist_hbm.at[wid])


@functools.partial(
    pl.kernel,
    out_type=jax.ShapeDtypeStruct((D, NP), jnp.float32),
    mesh=_mesh,
    compiler_params=_sc_params,
    scratch_types=[
        [pltpu.VMEM((NP,), jnp.float32)] * FPT,
        [pltpu.VMEM((NP,), jnp.float32)] * FPT,
        pltpu.VMEM((NP,), jnp.float32),
        pltpu.VMEM((2, CH), jnp.int32),
        pltpu.VMEM((16,), jnp.int32),
        pltpu.SemaphoreType.DMA,
        pltpu.SemaphoreType.DMA,
        pltpu.SemaphoreType.DMA,
    ],
)
def _prop_kernel(u0_hbm, edge_hbm, dinv_hbm, ns_hbm, out_hbm,
                 u_refs, acc_refs, dinv_t, ebuf, nsv, sem_a, sem_b, sem_m):
    wid = _wid()
    f0 = wid * FPT
    for f in range(FPT):
        pltpu.async_copy(u0_hbm.at[f0 + f], u_refs[f], sem_m)
    pltpu.async_copy(dinv_hbm, dinv_t, sem_m)
    pltpu.async_copy(ns_hbm, nsv, sem_m)
    for f in range(FPT):
        pltpu.make_async_copy(u0_hbm.at[f0 + f], u_refs[f], sem_m).wait()
    pltpu.make_async_copy(dinv_hbm, dinv_t, sem_m).wait()
    pltpu.make_async_copy(ns_hbm, nsv, sem_m).wait()
    nspk = jnp.max(nsv[...])
    nsteps = lax.bitwise_and(nspk, jnp.int32(0xFF))
    nch3 = lax.shift_right_logical(nspk, jnp.int32(8))

    def _issue(ci, buf, sem):
        pltpu.async_copy(edge_hbm.at[pl.ds(ci * CH, CH)], ebuf.at[buf], sem)

    def _wait(ci, buf, sem):
        pltpu.make_async_copy(
            edge_hbm.at[pl.ds(ci * CH, CH)], ebuf.at[buf], sem).wait()

    def _process(buf):
        @plsc.parallel_loop(0, CH, 16, unroll=8)
        def _(e):
            pk = ebuf[buf, pl.ds(e, 16)]
            s16, d16 = _unpack(pk)
            vs = [plsc.load_gather(u_refs[f], [s16]) for f in range(FPT)]
            for f in range(FPT):
                plsc.addupdate_scatter(acc_refs[f], [d16], vs[f])

    def _step(_, carry):
        @pl.loop(0, NP, step=16, unroll=4)
        def _(i):
            for f in range(FPT):
                acc_refs[f][pl.ds(i, 16)] = u_refs[f][pl.ds(i, 16)]

        _issue(0, 0, sem_a)

        @pl.loop(0, NCHUNK, step=2)
        def _(ci):
            _issue(ci + 1, 1, sem_b)
            _wait(ci, 0, sem_a)
            _process(0)

            @pl.when(ci + 2 < NCHUNK)
            def _():
                _issue(ci + 2, 0, sem_a)

            _wait(ci + 1, 1, sem_b)
            _process(1)

        @pl.loop(0, nch3)
        def _(c3):
            pltpu.sync_copy(edge_hbm.at[pl.ds(E_MAIN + c3 * CH, CH)],
                            ebuf.at[0])
            _process(0)

        @pl.loop(0, NP, step=16, unroll=4)
        def _(i):
            dv = dinv_t[pl.ds(i, 16)]
            for f in range(FPT):
                u_refs[f][pl.ds(i, 16)] = acc_refs[f][pl.ds(i, 16)] * dv

        return carry

    lax.fori_loop(0, nsteps, _step, 0)
    for f in range(FPT):
        pltpu.async_copy(u_refs[f], out_hbm.at[f0 + f], sem_m)
    for f in range(FPT):
        pltpu.make_async_copy(u_refs[f], out_hbm.at[f0 + f], sem_m).wait()


@functools.partial(
    pl.kernel,
    out_type=jax.ShapeDtypeStruct((CP, NP), jnp.float32),
    mesh=_mesh,
    compiler_params=_sc_params,
    scratch_types=[
        pltpu.VMEM((NP,), jnp.float32),
        pltpu.VMEM((NP,), jnp.float32),
        pltpu.VMEM((NP,), jnp.float32),
        pltpu.VMEM((2, CH), jnp.int32),
        pltpu.VMEM((16,), jnp.int32),
        pltpu.SemaphoreType.DMA,
        pltpu.SemaphoreType.DMA,
        pltpu.SemaphoreType.DMA,
    ],
)
def _final_kernel(z_hbm, edge_hbm, disq_hbm, ns_hbm, out_hbm,
                  z_t, acc_t, disq_t, ebuf, nsv, sem_a, sem_b, sem_m):
    wid = _wid()

    @pl.when(wid < CP)
    def _():
        pltpu.async_copy(z_hbm.at[wid], z_t, sem_m).wait()
        pltpu.async_copy(disq_hbm, disq_t, sem_m).wait()
        pltpu.async_copy(ns_hbm, nsv, sem_m).wait()
        nch3 = lax.shift_right_logical(jnp.max(nsv[...]), jnp.int32(8))

        @pl.loop(0, NP, step=16, unroll=4)
        def _(i):
            acc_t[pl.ds(i, 16)] = z_t[pl.ds(i, 16)]

        def _issue(ci, buf, sem):
            pltpu.async_copy(edge_hbm.at[pl.ds(ci * CH, CH)], ebuf.at[buf], sem)

        def _wait(ci, buf, sem):
            pltpu.make_async_copy(
                edge_hbm.at[pl.ds(ci * CH, CH)], ebuf.at[buf], sem).wait()

        def _process(buf):
            @plsc.parallel_loop(0, CH, 16, unroll=4)
            def _(e):
                pk = ebuf[buf, pl.ds(e, 16)]
                s16, d16 = _unpack(pk)
                v = plsc.load_gather(z_t, [s16])
                plsc.addupdate_scatter(acc_t, [d16], v)

        _issue(0, 0, sem_a)

        @pl.loop(0, NCHUNK, step=2)
        def _(ci):
            _issue(ci + 1, 1, sem_b)
            _wait(ci, 0, sem_a)
            _process(0)

            @pl.when(ci + 2 < NCHUNK)
            def _():
                _issue(ci + 2, 0, sem_a)

            _wait(ci + 1, 1, sem_b)
            _process(1)

        @pl.loop(0, nch3)
        def _(c3):
            pltpu.sync_copy(edge_hbm.at[pl.ds(E_MAIN + c3 * CH, CH)],
                            ebuf.at[0])
            _process(0)

        @pl.loop(0, NP, step=16, unroll=4)
        def _(i):
            acc_t[pl.ds(i, 16)] = acc_t[pl.ds(i, 16)] * disq_t[pl.ds(i, 16)]

        pltpu.sync_copy(acc_t, out_hbm.at[wid])


def _mm0_body(w_ref, x_ref, b_ref, o_ref):
    o_ref[...] = lax.dot_general(
        w_ref[...], x_ref[...], (((0,), (1,)), ((), ())),
        preferred_element_type=jnp.float32) + b_ref[...]


_mm0 = pl.pallas_call(
    _mm0_body,
    grid=(NP // COLB,),
    in_specs=[
        pl.BlockSpec((D, D), lambda i: (0, 0)),
        pl.BlockSpec((COLB, D), lambda i: (i, 0)),
        pl.BlockSpec((D, 1), lambda i: (0, 0)),
    ],
    out_specs=pl.BlockSpec((D, COLB), lambda i: (0, i)),
    out_shape=jax.ShapeDtypeStruct((D, NP), jnp.float32),
)


def _scale_body(h_ref, hist_ref, u0_ref, dinv_ref, dsq_ref, disq_ref):
    deg = 1.0 + jnp.sum(hist_ref[...], axis=0, keepdims=True)
    pos = deg > 0
    dinv_ref[...] = jnp.where(pos, 1.0 / deg, 0.0)
    sq = jnp.sqrt(deg)
    dsq_ref[...] = sq
    disq = jnp.where(pos, 1.0 / sq, 0.0)
    disq_ref[...] = disq
    u0_ref[...] = h_ref[...] * disq


_scale = pl.pallas_call(
    _scale_body,
    grid=(NP // COLB,),
    in_specs=[
        pl.BlockSpec((D, COLB), lambda i: (0, i)),
        pl.BlockSpec((NTILES, COLB), lambda i: (0, i)),
    ],
    out_specs=[
        pl.BlockSpec((D, COLB), lambda i: (0, i)),
        pl.BlockSpec((1, COLB), lambda i: (0, i)),
        pl.BlockSpec((1, COLB), lambda i: (0, i)),
        pl.BlockSpec((1, COLB), lambda i: (0, i)),
    ],
    out_shape=[
        jax.ShapeDtypeStruct((D, NP), jnp.float32),
        jax.ShapeDtypeStruct((1, NP), jnp.float32),
        jax.ShapeDtypeStruct((1, NP), jnp.float32),
        jax.ShapeDtypeStruct((1, NP), jnp.float32),
    ],
)


def _clf_body(wt_ref, u_ref, dsq_ref, disq_ref, bc_ref, z_ref):
    y = jnp.maximum(u_ref[...], 0.0) * dsq_ref[...]
    z = lax.dot_general(
        wt_ref[...], y, (((1,), (0,)), ((), ())),
        preferred_element_type=jnp.float32)
    z_ref[...] = (z + bc_ref[...]) * disq_ref[...]


_clf = pl.pallas_call(
    _clf_body,
    grid=(NP // COLB,),
    in_specs=[
        pl.BlockSpec((CP, D), lambda i: (0, 0)),
        pl.BlockSpec((D, COLB), lambda i: (0, i)),
        pl.BlockSpec((1, COLB), lambda i: (0, i)),
        pl.BlockSpec((1, COLB), lambda i: (0, i)),
        pl.BlockSpec((CP, 1), lambda i: (0, 0)),
    ],
    out_specs=pl.BlockSpec((CP, COLB), lambda i: (0, i)),
    out_shape=jax.ShapeDtypeStruct((CP, NP), jnp.float32),
)


def kernel(x, edge_index, prop_nums, W0, b0, Wc, bc):
    src = edge_index[0].astype(jnp.int32)
    dst = edge_index[1].astype(jnp.int32)
    pk = jnp.bitwise_or(src, jnp.left_shift(dst, 16))
    cnt = _cnt_kernel(pk)
    base = jnp.cumsum(cnt, axis=0) - cnt
    total = jnp.sum(cnt, axis=0)
    tot_sp = jnp.maximum(total - CAPV, 0)
    s_excl = jnp.cumsum(tot_sp) - tot_sp
    spill_base = s_excl[None, :] + jnp.maximum(base - CAPV, 0)
    epk_sc = _pos_kernel(pk, base, spill_base)
    tot2 = jnp.minimum(total, CAPV)
    n_spill = E_RAW - jnp.sum(tot2)
    n_ovf = jnp.maximum(n_spill - E_SPARE, 0)
    nch3 = (n_ovf + CH - 1) // CH
    realb = (jnp.arange(CAPV, dtype=jnp.int32)[None, :, None]
             < tot2.reshape(16, 16)[:, None, :]).reshape(E_BUCK)
    real_sp = jnp.arange(E_TOT - E_BUCK, dtype=jnp.int32) < n_spill
    real_full = jnp.concatenate([realb, real_sp])
    t = jnp.arange(E_TOT, dtype=jnp.int32)
    lpad = t % 16
    kpad = (t // REG) % 16
    spad = N_NODES + lpad
    dpad = N_NODES + ((lpad + kpad) % 16)
    pad_pk = jnp.bitwise_or(spad, jnp.left_shift(dpad, 16))
    epk = jnp.where(real_full, epk_sc, pad_pk)

    x_pad = jnp.pad(x, ((0, NP - N_NODES), (0, 0)))
    b0c = b0.reshape(D, 1)
    wct = jnp.pad(Wc, ((0, 0), (0, CP - C_OUT))).T
    bcp = jnp.pad(bc, (0, CP - C_OUT)).reshape(CP, 1)
    ns_arr = jnp.full((16,), prop_nums + nch3 * 256, jnp.int32)

    hist = _deg_kernel(epk)
    h0t = _mm0(W0, x_pad, b0c)
    u0, dinv, dsq, disq = _scale(h0t, hist)
    u30 = _prop_kernel(u0, epk, dinv.reshape(NP), ns_arr)
    z2 = _clf(wct, u30, dsq, disq, bcp)
    outt = _final_kernel(z2, epk, disq.reshape(NP), ns_arr)
    return outt[:C_OUT, :N_NODES].T
